# baseline (speedup 1.0000x reference)
import functools

import jax
import jax.numpy as jnp
from jax import lax
from jax.experimental import pallas as pl
from jax.experimental.pallas import tpu as pltpu
from jax.experimental.pallas import tpu_sc as plsc

N_NODES = 10000
N_EDGES = 160000
DIM_IN = 256
DIM_H = 512
DIM_E = 16
N_LAYERS = 4

SC_CORES = 2
SC_SUBCORES = 16
NW = SC_CORES * SC_SUBCORES
EPW = N_EDGES // NW
CHUNK = 40
NCHUNK = EPW // CHUNK
N_PAD = 10240
NPW = N_PAD // SC_SUBCORES
ZROWS = 128

XW = 128
BN = 2000
BE = 2000

@functools.cache
def _sc_mesh():
    return plsc.VectorSubcoreMesh(core_axis_name="c", subcore_axis_name="s",
                                  num_cores=SC_CORES,
                                  num_subcores=SC_SUBCORES)


def _sc_gather_body(a_hbm, b_hbm, x_hbm, rcv_hbm, src_hbm,
                    ar_out, br_out, xr_out, xs_out,
                    idxr_v, idxs_v,
                    buf_a0, buf_b0, buf_xr0, buf_xs0,
                    buf_a1, buf_b1, buf_xr1, buf_xs1,
                    buf_a2, buf_b2, buf_xr2, buf_xs2,
                    in_s0, in_s1, in_s2, out_s0, out_s1, out_s2):
    core = lax.axis_index("c")
    sub = lax.axis_index("s")
    wid = core * SC_SUBCORES + sub
    pltpu.sync_copy(rcv_hbm.at[wid], idxr_v)
    pltpu.sync_copy(src_hbm.at[wid], idxs_v)

    sets = ((buf_a0, buf_b0, buf_xr0, buf_xs0, in_s0, out_s0),
            (buf_a1, buf_b1, buf_xr1, buf_xs1, in_s1, out_s1),
            (buf_a2, buf_b2, buf_xr2, buf_xs2, in_s2, out_s2))

    def issue(j, b):
        ba, bb, bxr, bxs, ins, _ = sets[b]
        pltpu.async_copy(a_hbm.at[idxr_v.at[j]], ba, ins)
        pltpu.async_copy(b_hbm.at[idxs_v.at[j]], bb, ins)
        pltpu.async_copy(x_hbm.at[idxr_v.at[j]], bxr, ins)
        pltpu.async_copy(x_hbm.at[idxs_v.at[j]], bxs, ins)

    def drain_in(b):
        ba, bb, bxr, bxs, ins, _ = sets[b]
        pltpu.make_async_copy(a_hbm.at[pl.ds(0, CHUNK)], ba, ins).wait()
        pltpu.make_async_copy(b_hbm.at[pl.ds(0, CHUNK)], bb, ins).wait()
        pltpu.make_async_copy(x_hbm.at[pl.ds(0, CHUNK)], bxr, ins).wait()
        pltpu.make_async_copy(x_hbm.at[pl.ds(0, CHUNK)], bxs, ins).wait()

    def out_copies(j, b):
        ba, bb, bxr, bxs, _, outs = sets[b]
        base = wid * EPW + j * CHUNK
        pltpu.async_copy(ba, ar_out.at[pl.ds(base, CHUNK)], outs)
        pltpu.async_copy(bb, br_out.at[pl.ds(base, CHUNK)], outs)
        pltpu.async_copy(bxr, xr_out.at[pl.ds(base, CHUNK)], outs)
        pltpu.async_copy(bxs, xs_out.at[pl.ds(base, CHUNK)], outs)

    def drain_out(b):
        ba, bb, bxr, bxs, _, outs = sets[b]
        pltpu.make_async_copy(ba, ar_out.at[pl.ds(0, CHUNK)], outs).wait()
        pltpu.make_async_copy(bb, br_out.at[pl.ds(0, CHUNK)], outs).wait()
        pltpu.make_async_copy(bxr, xr_out.at[pl.ds(0, CHUNK)], outs).wait()
        pltpu.make_async_copy(bxs, xs_out.at[pl.ds(0, CHUNK)], outs).wait()

    issue(0, 0)
    issue(1, 1)
    issue(2, 2)

    def body(k, carry):
        for b in range(3):
            j = 3 * k + b
            drain_in(b)
            out_copies(j, b)
            drain_out(b)

            @pl.when(j + 3 < NCHUNK)
            def _():
                issue(j + 3, b)
        return carry

    lax.fori_loop(0, NCHUNK // 3, body, 0)
    for b in range(NCHUNK - 3 * (NCHUNK // 3)):
        j = 3 * (NCHUNK // 3) + b
        drain_in(b)
        out_copies(j, b)
        drain_out(b)


@functools.cache
def _sc_gather_fn():
    return pl.kernel(
        _sc_gather_body,
        out_type=[
            jax.ShapeDtypeStruct((N_EDGES, 256), jnp.uint32),
            jax.ShapeDtypeStruct((N_EDGES, 256), jnp.uint32),
            jax.ShapeDtypeStruct((N_EDGES, XW), jnp.float32),
            jax.ShapeDtypeStruct((N_EDGES, XW), jnp.float32),
        ],
        mesh=_sc_mesh(),
        scratch_types=[
            pltpu.VMEM((NCHUNK, CHUNK), jnp.int32),
            pltpu.VMEM((NCHUNK, CHUNK), jnp.int32),
            pltpu.VMEM((CHUNK, 256), jnp.uint32),
            pltpu.VMEM((CHUNK, 256), jnp.uint32),
            pltpu.VMEM((CHUNK, XW), jnp.float32),
            pltpu.VMEM((CHUNK, XW), jnp.float32),
            pltpu.VMEM((CHUNK, 256), jnp.uint32),
            pltpu.VMEM((CHUNK, 256), jnp.uint32),
            pltpu.VMEM((CHUNK, XW), jnp.float32),
            pltpu.VMEM((CHUNK, XW), jnp.float32),
            pltpu.VMEM((CHUNK, 256), jnp.uint32),
            pltpu.VMEM((CHUNK, 256), jnp.uint32),
            pltpu.VMEM((CHUNK, XW), jnp.float32),
            pltpu.VMEM((CHUNK, XW), jnp.float32),
            pltpu.SemaphoreType.DMA,
            pltpu.SemaphoreType.DMA,
            pltpu.SemaphoreType.DMA,
            pltpu.SemaphoreType.DMA,
            pltpu.SemaphoreType.DMA,
            pltpu.SemaphoreType.DMA,
        ],
    )


def _sc_gather(*args):
    return _sc_gather_fn()(*args)


def _sc_scatter_m_body(m0, m1, m2, m3, rc, zeros_hbm, rcv_hbm,
                       g0, g1, g2, g3, gx,
                       idx_v, mbuf0, mbuf1, mbuf2, mbuf3, mbuf4, acc,
                       in_s0, in_s1, in_s2, in_s3, in_s4,
                       add_s0, add_s1, add_s2, add_s3, add_s4):
    core = lax.axis_index("c")
    sub = lax.axis_index("s")
    wid = core * SC_SUBCORES + sub
    pltpu.sync_copy(rcv_hbm.at[wid], idx_v)

    stripe0 = sub * NPW
    sets = ((mbuf0, in_s0, add_s0), (mbuf1, in_s1, add_s1),
            (mbuf2, in_s2, add_s2), (mbuf3, in_s3, add_s3),
            (mbuf4, in_s4, add_s4))
    NS = 5

    for m_hbm, g_out in ((m0, g0), (m1, g1), (m2, g2), (m3, g3), (rc, gx)):
        for t in range(NPW // ZROWS):
            pltpu.sync_copy(zeros_hbm,
                            acc.at[pl.ds(stripe0 + t * ZROWS, ZROWS)])
        plsc.subcore_barrier()

        def issue_in(j, b, m_hbm=m_hbm):
            buf, ins, _ = sets[b]
            base = wid * EPW + j * CHUNK
            pltpu.async_copy(m_hbm.at[pl.ds(base, CHUNK)], buf, ins)

        def drain_in(b, m_hbm=m_hbm):
            buf, ins, _ = sets[b]
            pltpu.make_async_copy(m_hbm.at[pl.ds(0, CHUNK)], buf, ins).wait()

        def add_sync(j, b):
            buf, _, adds = sets[b]
            pltpu.async_copy(buf, acc.at[idx_v.at[j]], adds, add=True).wait()

        for b in range(NS):
            issue_in(b, b)

        def body(k, carry):
            for b in range(NS):
                j = NS * k + b
                drain_in(b)
                add_sync(j, b)

                @pl.when(j + NS < NCHUNK)
                def _():
                    issue_in(j + NS, b)
            return carry

        lax.fori_loop(0, NCHUNK // NS, body, 0)
        plsc.subcore_barrier()
        pltpu.sync_copy(acc.at[pl.ds(stripe0, NPW)],
                        g_out.at[core, pl.ds(stripe0, NPW)])
        plsc.subcore_barrier()


@functools.cache
def _sc_scatter_m_fn():
    return pl.kernel(
        _sc_scatter_m_body,
        out_type=[
            jax.ShapeDtypeStruct((SC_CORES, N_PAD, 128), jnp.float32),
            jax.ShapeDtypeStruct((SC_CORES, N_PAD, 128), jnp.float32),
            jax.ShapeDtypeStruct((SC_CORES, N_PAD, 128), jnp.float32),
            jax.ShapeDtypeStruct((SC_CORES, N_PAD, 128), jnp.float32),
            jax.ShapeDtypeStruct((SC_CORES, N_PAD, 128), jnp.float32),
        ],
        mesh=_sc_mesh(),
        scratch_types=[
            pltpu.VMEM((NCHUNK, CHUNK), jnp.int32),
            pltpu.VMEM((CHUNK, 128), jnp.float32),
            pltpu.VMEM((CHUNK, 128), jnp.float32),
            pltpu.VMEM((CHUNK, 128), jnp.float32),
            pltpu.VMEM((CHUNK, 128), jnp.float32),
            pltpu.VMEM((CHUNK, 128), jnp.float32),
            pltpu.VMEM_SHARED((N_PAD, 128), jnp.float32),
        ] + [pltpu.SemaphoreType.DMA] * 10,
    )


def _sc_scatter(m0, m1, m2, m3, rc, rcv2):
    zeros = jnp.zeros((ZROWS, 128), jnp.float32)
    return _sc_scatter_m_fn()(m0, m1, m2, m3, rc, zeros, rcv2)


def _mm(a, b):
    return jnp.dot(a, b, preferred_element_type=jnp.float32)


def _pack_half(x):
    lo = lax.bitcast_convert_type(
        x[:, :DIM_H // 2].astype(jnp.bfloat16), jnp.uint16).astype(jnp.uint32)
    hi = lax.bitcast_convert_type(
        x[:, DIM_H // 2:].astype(jnp.bfloat16), jnp.uint16).astype(jnp.uint32)
    return lo | (hi << 16)


def _unpack_half(w):
    lo = lax.bitcast_convert_type(w.astype(jnp.uint16), jnp.bfloat16)
    hi = lax.bitcast_convert_type((w >> 16).astype(jnp.uint16), jnp.bfloat16)
    return lo.astype(jnp.float32), hi.astype(jnp.float32)


def _prologue_body(feat, w_in, b_in, we1i, we1j, h0_o, a_o, b_o):
    h0 = _mm(feat[...], w_in[...]) + b_in[...]
    h0_o[...] = h0
    a_o[...] = _pack_half(_mm(h0, we1i[...]))
    b_o[...] = _pack_half(_mm(h0, we1j[...]))


def _tc_prologue(feat, w_in, b_in, we1i, we1j):
    grid = (N_NODES // BN,)
    row = pl.BlockSpec((BN, DIM_IN), lambda i: (i, 0))
    out = pl.BlockSpec((BN, DIM_H), lambda i: (i, 0))
    outT = pl.BlockSpec((BN, DIM_H // 2), lambda i: (i, 0))
    full = lambda shape: pl.BlockSpec(shape, lambda i: (0,) * len(shape))
    return pl.pallas_call(
        _prologue_body,
        grid=grid,
        in_specs=[row, full((DIM_IN, DIM_H)), full((1, DIM_H)),
                  full((DIM_H, DIM_H)), full((DIM_H, DIM_H))],
        out_specs=[out, outT, outT],
        out_shape=[jax.ShapeDtypeStruct((N_NODES, DIM_H), jnp.float32),
                   jax.ShapeDtypeStruct((N_NODES, DIM_H // 2), jnp.uint32),
                   jax.ShapeDtypeStruct((N_NODES, DIM_H // 2), jnp.uint32)],
    )(feat, w_in, b_in, we1i, we1j)


def _edge_body(ar, br, xr, xs, ea, we1e, wd, be1, we2, be2, wx, bx,
               m0_o, m1_o, m2_o, m3_o, rc_o):
    alo, ahi = _unpack_half(ar[...])
    blo, bhi = _unpack_half(br[...])
    pre = jnp.concatenate([alo + blo, ahi + bhi], axis=1)
    rel = xr[...] - xs[...]
    d2 = jnp.sum(rel * rel, axis=1, keepdims=True)
    t = pre + d2 * wd[...] + _mm(ea[...], we1e[...]) + be1[...]
    t = jax.nn.silu(t).astype(jnp.bfloat16)
    m = jax.nn.silu(_mm(t, we2[...].astype(jnp.bfloat16)) + be2[...])
    cw = jnp.tanh(jnp.sum(m * wx[...], axis=1, keepdims=True) + bx[...])
    col = lax.broadcasted_iota(jnp.int32, (BE, XW), 1)
    rc = rel * cw + jnp.where(col == 3, 1.0, 0.0)
    m0_o[...] = m[:, 0:128]
    m1_o[...] = m[:, 128:256]
    m2_o[...] = m[:, 256:384]
    m3_o[...] = m[:, 384:512]
    rc_o[...] = rc


def _tc_edge(ar, br, xr, xs, ea, we1e, wd, be1, we2, be2, wx, bx):
    grid = (N_EDGES // BE,)
    rowH = pl.BlockSpec((BE, DIM_H // 2), lambda i: (i, 0))
    rowE = pl.BlockSpec((BE, DIM_E), lambda i: (i, 0))
    rowX = pl.BlockSpec((BE, XW), lambda i: (i, 0))
    full = lambda shape: pl.BlockSpec(shape, lambda i: (0,) * len(shape))
    out128 = pl.BlockSpec((BE, 128), lambda i: (i, 0))
    return pl.pallas_call(
        _edge_body,
        grid=grid,
        in_specs=[rowH, rowH, rowX, rowX, rowE,
                  full((DIM_E, DIM_H)), full((1, DIM_H)), full((1, DIM_H)),
                  full((DIM_H, DIM_H)), full((1, DIM_H)), full((1, DIM_H)),
                  full((1, 1))],
        out_specs=[out128, out128, out128, out128, out128],
        out_shape=[jax.ShapeDtypeStruct((N_EDGES, 128), jnp.float32)] * 5,
    )(ar, br, xr, xs, ea, we1e, wd, be1, we2, be2, wx, bx)


def _node_common(h_ref, g0, g1, g2, g3, gx, wh1h, wh1a, bh1, wh2, bh2):
    h = h_ref[...]
    agg = jnp.concatenate(
        [g0[0] + g0[1], g1[0] + g1[1], g2[0] + g2[1], g3[0] + g3[1]], axis=1)
    xa = gx[0] + gx[1]
    deg = xa[:, 3:4]

    invd = 1.0 / jnp.maximum(deg, 1.0)
    agg = agg * invd
    u = jax.nn.silu(_mm(h, wh1h[...]) + _mm(agg, wh1a[...]) + bh1[...])
    h_new = h + _mm(u, wh2[...]) + bh2[...]
    return h_new, xa, invd


def _node_body(h_ref, xp, g0, g1, g2, g3, gx,
               wh1h, wh1a, bh1, wh2, bh2, we1i, we1j,
               h_o, x_o, a_o, b_o):
    h_new, xa, invd = _node_common(h_ref, g0, g1, g2, g3, gx,
                                   wh1h, wh1a, bh1, wh2, bh2)
    col = lax.broadcasted_iota(jnp.int32, (BN, XW), 1)
    mask3 = jnp.where(col < 3, 1.0, 0.0)
    x_o[...] = xp[...] + xa * invd * mask3
    h_o[...] = h_new
    a_o[...] = _pack_half(_mm(h_new, we1i[...]))
    b_o[...] = _pack_half(_mm(h_new, we1j[...]))


def _final_body(h_ref, xp, g0, g1, g2, g3, gx,
                wh1h, wh1a, bh1, wh2, bh2, h_o, gp_o):
    h_new, _, _ = _node_common(h_ref, g0, g1, g2, g3, gx,
                               wh1h, wh1a, bh1, wh2, bh2)
    h_o[...] = h_new

    @pl.when(pl.program_id(0) == 0)
    def _():
        gp_o[...] = jnp.zeros_like(gp_o)

    gp_o[...] += jnp.sum(h_new, axis=0, keepdims=True) / N_NODES


def _node_specs():
    rowH = pl.BlockSpec((BN, DIM_H), lambda i: (i, 0))
    rowE = pl.BlockSpec((BN, XW), lambda i: (i, 0))
    g128 = pl.BlockSpec((SC_CORES, BN, 128), lambda i: (0, i, 0))
    gE = g128
    full = lambda shape: pl.BlockSpec(shape, lambda i: (0,) * len(shape))
    w = full((DIM_H, DIM_H))
    b = full((1, DIM_H))
    return rowH, rowE, g128, gE, w, b


def _tc_node(h, xp, gs, gx, wh1h, wh1a, bh1, wh2, bh2, we1i, we1j):
    rowH, rowE, g128, gE, w, b = _node_specs()
    outT = pl.BlockSpec((BN, DIM_H // 2), lambda i: (i, 0))
    return pl.pallas_call(
        _node_body,
        grid=(N_NODES // BN,),
        in_specs=[rowH, rowE, g128, g128, g128, g128, gE, w, w, b, w, b, w, w],
        out_specs=[rowH, rowE, outT, outT],
        out_shape=[
            jax.ShapeDtypeStruct((N_NODES, DIM_H), jnp.float32),
            jax.ShapeDtypeStruct((N_NODES, XW), jnp.float32),
            jax.ShapeDtypeStruct((N_NODES, DIM_H // 2), jnp.uint32),
            jax.ShapeDtypeStruct((N_NODES, DIM_H // 2), jnp.uint32),
        ],
    )(h, xp, *gs, gx, wh1h, wh1a, bh1, wh2, bh2, we1i, we1j)


def _tc_final(h, xp, gs, gx, wh1h, wh1a, bh1, wh2, bh2):
    rowH, rowE, g128, gE, w, b = _node_specs()
    gp = pl.BlockSpec((1, DIM_H), lambda i: (0, 0))
    return pl.pallas_call(
        _final_body,
        grid=(N_NODES // BN,),
        in_specs=[rowH, rowE, g128, g128, g128, g128, gE, w, w, b, w, b],
        out_specs=[rowH, gp],
        out_shape=[
            jax.ShapeDtypeStruct((N_NODES, DIM_H), jnp.float32),
            jax.ShapeDtypeStruct((1, DIM_H), jnp.float32),
        ],
    )(h, xp, *gs, gx, wh1h, wh1a, bh1, wh2, bh2)


def kernel(protein_pos, protein_atom_feature, pp_edge_index, pp_edge_attr,
           params):
    src = pp_edge_index[0].astype(jnp.int32)
    rcv = pp_edge_index[1].astype(jnp.int32)
    rcv2 = rcv.reshape(NW, NCHUNK, CHUNK)
    src2 = src.reshape(NW, NCHUNK, CHUNK)
    xp = jnp.pad(protein_pos.astype(jnp.float32), ((0, 0), (0, XW - 3)))

    layers = params['layers']

    def wsplit(p):
        we1 = p['We1']
        return (we1[0:DIM_H], we1[DIM_H:2 * DIM_H],
                we1[2 * DIM_H:2 * DIM_H + 1],
                we1[2 * DIM_H + 1:])

    we1i0, we1j0, _, _ = wsplit(layers[0])
    h, a, b = _tc_prologue(
        protein_atom_feature, params['W_in'], params['b_in'].reshape(1, DIM_H),
        we1i0, we1j0)

    for l in range(N_LAYERS):
        p = layers[l]
        _, _, wd, we1e = wsplit(p)
        ar, br, xr, xs = _sc_gather(a, b, xp, rcv2, src2)
        m0, m1, m2, m3, rc = _tc_edge(
            ar, br, xr, xs, pp_edge_attr,
            we1e, wd, p['be1'].reshape(1, DIM_H),
            p['We2'], p['be2'].reshape(1, DIM_H),
            p['Wx'].reshape(1, DIM_H), p['bx'].reshape(1, 1))
        g0, g1, g2, g3, gx = _sc_scatter(m0, m1, m2, m3, rc, rcv2)
        wh1 = p['Wh1']
        if l < N_LAYERS - 1:
            we1i_n, we1j_n, _, _ = wsplit(layers[l + 1])
            h, xp, a, b = _tc_node(
                h, xp, (g0, g1, g2, g3), gx,
                wh1[0:DIM_H], wh1[DIM_H:], p['bh1'].reshape(1, DIM_H),
                p['Wh2'], p['bh2'].reshape(1, DIM_H), we1i_n, we1j_n)
        else:
            h, gp = _tc_final(
                h, xp, (g0, g1, g2, g3), gx,
                wh1[0:DIM_H], wh1[DIM_H:], p['bh1'].reshape(1, DIM_H),
                p['Wh2'], p['bh2'].reshape(1, DIM_H))

    return (h, gp.reshape(DIM_H))


# speedup vs baseline: 1.0909x; 1.0909x over previous
import functools

import jax
import jax.numpy as jnp
from jax import lax
from jax.experimental import pallas as pl
from jax.experimental.pallas import tpu as pltpu
from jax.experimental.pallas import tpu_sc as plsc

N_NODES = 10000
N_EDGES = 160000
DIM_IN = 256
DIM_H = 512
DIM_E = 16
N_LAYERS = 4

SC_CORES = 2
SC_SUBCORES = 16
NW = SC_CORES * SC_SUBCORES
EPW = N_EDGES // NW
CHUNK = 40
NCHUNK = EPW // CHUNK
N_PAD = 10240
NPW = N_PAD // SC_SUBCORES
ZROWS = 128

XW = 128
BN = 1000
BE = 1000

@functools.cache
def _sc_mesh():
    return plsc.VectorSubcoreMesh(core_axis_name="c", subcore_axis_name="s",
                                  num_cores=SC_CORES,
                                  num_subcores=SC_SUBCORES)


def _sc_gather_body(a_hbm, b_hbm, xpl_hbm, rcv_hbm, src_hbm,
                    ar_out, br_out, xrel_out,
                    idxr_v, idxs_v, xpl_v,
                    buf_a0, buf_b0, buf_r0,
                    buf_a1, buf_b1, buf_r1,
                    in_s0, in_s1, out_s0, out_s1):
    core = lax.axis_index("c")
    sub = lax.axis_index("s")
    wid = core * SC_SUBCORES + sub
    pltpu.sync_copy(rcv_hbm.at[wid], idxr_v)
    pltpu.sync_copy(src_hbm.at[wid], idxs_v)
    pltpu.sync_copy(xpl_hbm, xpl_v)

    sets = ((buf_a0, buf_b0, buf_r0, in_s0, out_s0),
            (buf_a1, buf_b1, buf_r1, in_s1, out_s1))

    zero16 = jnp.zeros((16,), jnp.float32)

    def zrel(i, carry):
        for _, _, br_, _, _ in sets:
            br_[i, pl.ds(0, 16)] = zero16
        return carry

    lax.fori_loop(0, CHUNK, zrel, 0)

    iota = lax.broadcasted_iota(jnp.int32, (16,), 0)
    tail_mask = iota >= 8

    def issue(j, b):
        ba, bb, _, ins, _ = sets[b]
        pltpu.async_copy(a_hbm.at[idxr_v.at[j]], ba, ins)
        pltpu.async_copy(b_hbm.at[idxs_v.at[j]], bb, ins)

    def drain_in(b):
        ba, bb, _, ins, _ = sets[b]
        pltpu.make_async_copy(a_hbm.at[pl.ds(0, CHUNK)], ba, ins).wait()
        pltpu.make_async_copy(b_hbm.at[pl.ds(0, CHUNK)], bb, ins).wait()

    def rel_compute(j, b):
        _, _, br_, _, _ = sets[b]
        for off, mask in ((0, None), (16, None), (24, tail_mask)):
            ivr = idxr_v[j, pl.ds(off, 16)]
            ivs = idxs_v[j, pl.ds(off, 16)]
            rows = iota + off
            d2 = zero16
            for c in range(3):
                cc = jnp.full((16,), c, jnp.int32)
                xr = plsc.load_gather(xpl_v, [cc, ivr])
                xs = plsc.load_gather(xpl_v, [cc, ivs])
                rel = xr - xs
                d2 = d2 + rel * rel
                plsc.store_scatter(br_, [rows, cc], rel, mask=mask)
            plsc.store_scatter(br_, [rows, jnp.full((16,), 3, jnp.int32)],
                               d2, mask=mask)

    def out_copies(j, b):
        ba, bb, br_, _, outs = sets[b]
        base = wid * EPW + j * CHUNK
        pltpu.async_copy(ba, ar_out.at[pl.ds(base, CHUNK)], outs)
        pltpu.async_copy(bb, br_out.at[pl.ds(base, CHUNK)], outs)
        pltpu.async_copy(br_, xrel_out.at[pl.ds(base, CHUNK)], outs)

    def drain_out(b):
        ba, bb, br_, _, outs = sets[b]
        pltpu.make_async_copy(ba, ar_out.at[pl.ds(0, CHUNK)], outs).wait()
        pltpu.make_async_copy(bb, br_out.at[pl.ds(0, CHUNK)], outs).wait()
        pltpu.make_async_copy(br_, xrel_out.at[pl.ds(0, CHUNK)], outs).wait()

    issue(0, 0)
    issue(1, 1)

    def body(k, carry):
        for b in range(2):
            j = 2 * k + b
            rel_compute(j, b)
            drain_in(b)
            out_copies(j, b)
            drain_out(b)

            @pl.when(j + 2 < NCHUNK)
            def _():
                issue(j + 2, b)
        return carry

    lax.fori_loop(0, NCHUNK // 2, body, 0)
    for b in range(NCHUNK - 2 * (NCHUNK // 2)):
        j = 2 * (NCHUNK // 2) + b
        rel_compute(j, b)
        drain_in(b)
        out_copies(j, b)
        drain_out(b)


@functools.cache
def _sc_gather_fn():
    return pl.kernel(
        _sc_gather_body,
        out_type=[
            jax.ShapeDtypeStruct((N_EDGES, 256), jnp.uint32),
            jax.ShapeDtypeStruct((N_EDGES, 256), jnp.uint32),
            jax.ShapeDtypeStruct((N_EDGES, DIM_E), jnp.float32),
        ],
        mesh=_sc_mesh(),
        scratch_types=[
            pltpu.VMEM((NCHUNK, CHUNK), jnp.int32),
            pltpu.VMEM((NCHUNK, CHUNK), jnp.int32),
            pltpu.VMEM((3, N_NODES), jnp.float32),
            pltpu.VMEM((CHUNK, 256), jnp.uint32),
            pltpu.VMEM((CHUNK, 256), jnp.uint32),
            pltpu.VMEM((CHUNK, DIM_E), jnp.float32),
            pltpu.VMEM((CHUNK, 256), jnp.uint32),
            pltpu.VMEM((CHUNK, 256), jnp.uint32),
            pltpu.VMEM((CHUNK, DIM_E), jnp.float32),
            pltpu.SemaphoreType.DMA,
            pltpu.SemaphoreType.DMA,
            pltpu.SemaphoreType.DMA,
            pltpu.SemaphoreType.DMA,
        ],
        compiler_params=pltpu.CompilerParams(needs_layout_passes=False),
    )


def _sc_gather(*args):
    return _sc_gather_fn()(*args)


def _sc_scatter_m_body(m0, m1, m2, m3, rc, zeros_hbm, rcv_hbm,
                       g0, g1, g2, g3, gx,
                       idx_v, mbuf0, mbuf1, mbuf2, mbuf3, mbuf4, acc,
                       in_s0, in_s1, in_s2, in_s3, in_s4,
                       add_s0, add_s1, add_s2, add_s3, add_s4):
    core = lax.axis_index("c")
    sub = lax.axis_index("s")
    wid = core * SC_SUBCORES + sub
    pltpu.sync_copy(rcv_hbm.at[wid], idx_v)

    stripe0 = sub * NPW
    sets = ((mbuf0, in_s0, add_s0), (mbuf1, in_s1, add_s1),
            (mbuf2, in_s2, add_s2), (mbuf3, in_s3, add_s3),
            (mbuf4, in_s4, add_s4))
    NS = 5

    for m_hbm, g_out in ((m0, g0), (m1, g1), (m2, g2), (m3, g3), (rc, gx)):
        for t in range(NPW // ZROWS):
            pltpu.sync_copy(zeros_hbm,
                            acc.at[pl.ds(stripe0 + t * ZROWS, ZROWS)])
        plsc.subcore_barrier()

        def issue_in(j, b, m_hbm=m_hbm):
            buf, ins, _ = sets[b]
            base = wid * EPW + j * CHUNK
            pltpu.async_copy(m_hbm.at[pl.ds(base, CHUNK)], buf, ins)

        def drain_in(b, m_hbm=m_hbm):
            buf, ins, _ = sets[b]
            pltpu.make_async_copy(m_hbm.at[pl.ds(0, CHUNK)], buf, ins).wait()

        def add_sync(j, b):
            buf, _, adds = sets[b]
            pltpu.async_copy(buf, acc.at[idx_v.at[j]], adds, add=True).wait()

        for b in range(NS):
            issue_in(b, b)

        def body(k, carry):
            for b in range(NS):
                j = NS * k + b
                drain_in(b)
                add_sync(j, b)

                @pl.when(j + NS < NCHUNK)
                def _():
                    issue_in(j + NS, b)
            return carry

        lax.fori_loop(0, NCHUNK // NS, body, 0)
        plsc.subcore_barrier()
        pltpu.sync_copy(acc.at[pl.ds(stripe0, NPW)],
                        g_out.at[core, pl.ds(stripe0, NPW)])
        plsc.subcore_barrier()


@functools.cache
def _sc_scatter_m_fn():
    return pl.kernel(
        _sc_scatter_m_body,
        out_type=[
            jax.ShapeDtypeStruct((SC_CORES, N_PAD, 128), jnp.float32),
            jax.ShapeDtypeStruct((SC_CORES, N_PAD, 128), jnp.float32),
            jax.ShapeDtypeStruct((SC_CORES, N_PAD, 128), jnp.float32),
            jax.ShapeDtypeStruct((SC_CORES, N_PAD, 128), jnp.float32),
            jax.ShapeDtypeStruct((SC_CORES, N_PAD, 128), jnp.float32),
        ],
        mesh=_sc_mesh(),
        scratch_types=[
            pltpu.VMEM((NCHUNK, CHUNK), jnp.int32),
            pltpu.VMEM((CHUNK, 128), jnp.float32),
            pltpu.VMEM((CHUNK, 128), jnp.float32),
            pltpu.VMEM((CHUNK, 128), jnp.float32),
            pltpu.VMEM((CHUNK, 128), jnp.float32),
            pltpu.VMEM((CHUNK, 128), jnp.float32),
            pltpu.VMEM_SHARED((N_PAD, 128), jnp.float32),
        ] + [pltpu.SemaphoreType.DMA] * 10,
    )


def _sc_scatter(m0, m1, m2, m3, rc, rcv2):
    zeros = jnp.zeros((ZROWS, 128), jnp.float32)
    return _sc_scatter_m_fn()(m0, m1, m2, m3, rc, zeros, rcv2)


def _mm(a, b):
    return jnp.dot(a, b, preferred_element_type=jnp.float32)


def _pack_half(x):
    lo = lax.bitcast_convert_type(
        x[:, :DIM_H // 2].astype(jnp.bfloat16), jnp.uint16).astype(jnp.uint32)
    hi = lax.bitcast_convert_type(
        x[:, DIM_H // 2:].astype(jnp.bfloat16), jnp.uint16).astype(jnp.uint32)
    return lo | (hi << 16)


def _unpack_half(w):
    lo = lax.bitcast_convert_type(w.astype(jnp.uint16), jnp.bfloat16)
    hi = lax.bitcast_convert_type((w >> 16).astype(jnp.uint16), jnp.bfloat16)
    return lo.astype(jnp.float32), hi.astype(jnp.float32)


def _prologue_body(feat, w_in, b_in, we1i, we1j, h0_o, a_o, b_o):
    h0 = _mm(feat[...], w_in[...]) + b_in[...]
    h0_o[...] = h0
    a_o[...] = _pack_half(_mm(h0, we1i[...]))
    b_o[...] = _pack_half(_mm(h0, we1j[...]))


def _tc_prologue(feat, w_in, b_in, we1i, we1j):
    grid = (N_NODES // BN,)
    row = pl.BlockSpec((BN, DIM_IN), lambda i: (i, 0))
    out = pl.BlockSpec((BN, DIM_H), lambda i: (i, 0))
    outT = pl.BlockSpec((BN, DIM_H // 2), lambda i: (i, 0))
    full = lambda shape: pl.BlockSpec(shape, lambda i: (0,) * len(shape))
    return pl.pallas_call(
        _prologue_body,
        grid=grid,
        in_specs=[row, full((DIM_IN, DIM_H)), full((1, DIM_H)),
                  full((DIM_H, DIM_H)), full((DIM_H, DIM_H))],
        out_specs=[out, outT, outT],
        out_shape=[jax.ShapeDtypeStruct((N_NODES, DIM_H), jnp.float32),
                   jax.ShapeDtypeStruct((N_NODES, DIM_H // 2), jnp.uint32),
                   jax.ShapeDtypeStruct((N_NODES, DIM_H // 2), jnp.uint32)],
    )(feat, w_in, b_in, we1i, we1j)


def _edge_body(ar, br, xrel, ea, we1e, wd, be1, we2, be2, wx, bx,
               m0_o, m1_o, m2_o, m3_o, rc_o):
    alo, ahi = _unpack_half(ar[...])
    blo, bhi = _unpack_half(br[...])
    pre = jnp.concatenate([alo + blo, ahi + bhi], axis=1)
    xre = xrel[...]
    d2 = xre[:, 3:4]
    t = pre + d2 * wd[...] + _mm(ea[...], we1e[...]) + be1[...]
    t = jax.nn.silu(t).astype(jnp.bfloat16)
    m = jax.nn.silu(_mm(t, we2[...].astype(jnp.bfloat16)) + be2[...])
    cw = jnp.tanh(jnp.sum(m * wx[...], axis=1, keepdims=True) + bx[...])
    col = lax.broadcasted_iota(jnp.int32, (BE, DIM_E), 1)
    rel = jnp.where(col < 3, xre, 0.0)
    rc16 = rel * cw + jnp.where(col == 3, 1.0, 0.0)
    rc = jnp.concatenate([rc16, jnp.zeros((BE, XW - DIM_E), jnp.float32)],
                         axis=1)
    m0_o[...] = m[:, 0:128]
    m1_o[...] = m[:, 128:256]
    m2_o[...] = m[:, 256:384]
    m3_o[...] = m[:, 384:512]
    rc_o[...] = rc


def _tc_edge(ar, br, xrel, ea, we1e, wd, be1, we2, be2, wx, bx):
    grid = (N_EDGES // BE,)
    rowH = pl.BlockSpec((BE, DIM_H // 2), lambda i: (i, 0))
    rowE = pl.BlockSpec((BE, DIM_E), lambda i: (i, 0))
    rowX = pl.BlockSpec((BE, XW), lambda i: (i, 0))
    full = lambda shape: pl.BlockSpec(shape, lambda i: (0,) * len(shape))
    out128 = pl.BlockSpec((BE, 128), lambda i: (i, 0))
    return pl.pallas_call(
        _edge_body,
        grid=grid,
        in_specs=[rowH, rowH, rowE, rowE,
                  full((DIM_E, DIM_H)), full((1, DIM_H)), full((1, DIM_H)),
                  full((DIM_H, DIM_H)), full((1, DIM_H)), full((1, DIM_H)),
                  full((1, 1))],
        out_specs=[out128, out128, out128, out128, out128],
        out_shape=[jax.ShapeDtypeStruct((N_EDGES, 128), jnp.float32)] * 5,
    )(ar, br, xrel, ea, we1e, wd, be1, we2, be2, wx, bx)


def _node_common(h_ref, g0, g1, g2, g3, gx, wh1h, wh1a, bh1, wh2, bh2):
    h = h_ref[...]
    agg = jnp.concatenate(
        [g0[0] + g0[1], g1[0] + g1[1], g2[0] + g2[1], g3[0] + g3[1]], axis=1)
    xa = gx[0] + gx[1]
    deg = xa[:, 3:4]

    invd = 1.0 / jnp.maximum(deg, 1.0)
    agg = agg * invd
    u = jax.nn.silu(_mm(h, wh1h[...]) + _mm(agg, wh1a[...]) + bh1[...])
    h_new = h + _mm(u, wh2[...]) + bh2[...]
    return h_new, xa, invd


def _node_body(h_ref, xp, g0, g1, g2, g3, gx,
               wh1h, wh1a, bh1, wh2, bh2, we1i, we1j,
               h_o, x_o, a_o, b_o):
    h_new, xa, invd = _node_common(h_ref, g0, g1, g2, g3, gx,
                                   wh1h, wh1a, bh1, wh2, bh2)
    col = lax.broadcasted_iota(jnp.int32, (BN, XW), 1)
    mask3 = jnp.where(col < 3, 1.0, 0.0)
    x_o[...] = xp[...] + xa * invd * mask3
    h_o[...] = h_new
    a_o[...] = _pack_half(_mm(h_new, we1i[...]))
    b_o[...] = _pack_half(_mm(h_new, we1j[...]))


def _final_body(h_ref, xp, g0, g1, g2, g3, gx,
                wh1h, wh1a, bh1, wh2, bh2, h_o, gp_o):
    h_new, _, _ = _node_common(h_ref, g0, g1, g2, g3, gx,
                               wh1h, wh1a, bh1, wh2, bh2)
    h_o[...] = h_new

    @pl.when(pl.program_id(0) == 0)
    def _():
        gp_o[...] = jnp.zeros_like(gp_o)

    gp_o[...] += jnp.sum(h_new, axis=0, keepdims=True) / N_NODES


def _node_specs():
    rowH = pl.BlockSpec((BN, DIM_H), lambda i: (i, 0))
    rowE = pl.BlockSpec((BN, XW), lambda i: (i, 0))
    g128 = pl.BlockSpec((SC_CORES, BN, 128), lambda i: (0, i, 0))
    gE = g128
    full = lambda shape: pl.BlockSpec(shape, lambda i: (0,) * len(shape))
    w = full((DIM_H, DIM_H))
    b = full((1, DIM_H))
    return rowH, rowE, g128, gE, w, b


def _tc_node(h, xp, gs, gx, wh1h, wh1a, bh1, wh2, bh2, we1i, we1j):
    rowH, rowE, g128, gE, w, b = _node_specs()
    outT = pl.BlockSpec((BN, DIM_H // 2), lambda i: (i, 0))
    return pl.pallas_call(
        _node_body,
        grid=(N_NODES // BN,),
        in_specs=[rowH, rowE, g128, g128, g128, g128, gE, w, w, b, w, b, w, w],
        out_specs=[rowH, rowE, outT, outT],
        out_shape=[
            jax.ShapeDtypeStruct((N_NODES, DIM_H), jnp.float32),
            jax.ShapeDtypeStruct((N_NODES, XW), jnp.float32),
            jax.ShapeDtypeStruct((N_NODES, DIM_H // 2), jnp.uint32),
            jax.ShapeDtypeStruct((N_NODES, DIM_H // 2), jnp.uint32),
        ],
    )(h, xp, *gs, gx, wh1h, wh1a, bh1, wh2, bh2, we1i, we1j)


def _tc_final(h, xp, gs, gx, wh1h, wh1a, bh1, wh2, bh2):
    rowH, rowE, g128, gE, w, b = _node_specs()
    gp = pl.BlockSpec((1, DIM_H), lambda i: (0, 0))
    return pl.pallas_call(
        _final_body,
        grid=(N_NODES // BN,),
        in_specs=[rowH, rowE, g128, g128, g128, g128, gE, w, w, b, w, b],
        out_specs=[rowH, gp],
        out_shape=[
            jax.ShapeDtypeStruct((N_NODES, DIM_H), jnp.float32),
            jax.ShapeDtypeStruct((1, DIM_H), jnp.float32),
        ],
    )(h, xp, *gs, gx, wh1h, wh1a, bh1, wh2, bh2)


def kernel(protein_pos, protein_atom_feature, pp_edge_index, pp_edge_attr,
           params):
    src = pp_edge_index[0].astype(jnp.int32)
    rcv = pp_edge_index[1].astype(jnp.int32)
    rcv2 = rcv.reshape(NW, NCHUNK, CHUNK)
    src2 = src.reshape(NW, NCHUNK, CHUNK)
    xp = jnp.pad(protein_pos.astype(jnp.float32), ((0, 0), (0, XW - 3)))

    layers = params['layers']

    def wsplit(p):
        we1 = p['We1']
        return (we1[0:DIM_H], we1[DIM_H:2 * DIM_H],
                we1[2 * DIM_H:2 * DIM_H + 1],
                we1[2 * DIM_H + 1:])

    we1i0, we1j0, _, _ = wsplit(layers[0])
    h, a, b = _tc_prologue(
        protein_atom_feature, params['W_in'], params['b_in'].reshape(1, DIM_H),
        we1i0, we1j0)

    for l in range(N_LAYERS):
        p = layers[l]
        _, _, wd, we1e = wsplit(p)
        xpl = jnp.transpose(xp[:, :3])
        ar, br, xrel = _sc_gather(a, b, xpl, rcv2, src2)
        m0, m1, m2, m3, rc = _tc_edge(
            ar, br, xrel, pp_edge_attr,
            we1e, wd, p['be1'].reshape(1, DIM_H),
            p['We2'], p['be2'].reshape(1, DIM_H),
            p['Wx'].reshape(1, DIM_H), p['bx'].reshape(1, 1))
        g0, g1, g2, g3, gx = _sc_scatter(m0, m1, m2, m3, rc, rcv2)
        wh1 = p['Wh1']
        if l < N_LAYERS - 1:
            we1i_n, we1j_n, _, _ = wsplit(layers[l + 1])
            h, xp, a, b = _tc_node(
                h, xp, (g0, g1, g2, g3), gx,
                wh1[0:DIM_H], wh1[DIM_H:], p['bh1'].reshape(1, DIM_H),
                p['Wh2'], p['bh2'].reshape(1, DIM_H), we1i_n, we1j_n)
        else:
            h, gp = _tc_final(
                h, xp, (g0, g1, g2, g3), gx,
                wh1[0:DIM_H], wh1[DIM_H:], p['bh1'].reshape(1, DIM_H),
                p['Wh2'], p['bh2'].reshape(1, DIM_H))

    return (h, gp.reshape(DIM_H))


# speedup vs baseline: 1.0917x; 1.0007x over previous
import functools

import jax
import jax.numpy as jnp
from jax import lax
from jax.experimental import pallas as pl
from jax.experimental.pallas import tpu as pltpu
from jax.experimental.pallas import tpu_sc as plsc

N_NODES = 10000
N_EDGES = 160000
DIM_IN = 256
DIM_H = 512
DIM_E = 16
N_LAYERS = 4

SC_CORES = 2
SC_SUBCORES = 16
NW = SC_CORES * SC_SUBCORES
EPW = N_EDGES // NW
CHUNK = 40
NCHUNK = EPW // CHUNK
N_PAD = 10240
NPW = N_PAD // SC_SUBCORES
ZROWS = 128

XW = 128
BN = 1000
BE = 1000

@functools.cache
def _sc_mesh():
    return plsc.VectorSubcoreMesh(core_axis_name="c", subcore_axis_name="s",
                                  num_cores=SC_CORES,
                                  num_subcores=SC_SUBCORES)


def _sc_gather_body(a_hbm, b_hbm, xpl_hbm, rcv_hbm, src_hbm,
                    ar_out, br_out, xrel_out,
                    idxr_v, idxs_v, xpl_v,
                    buf_a0, buf_b0, buf_r0,
                    buf_a1, buf_b1, buf_r1,
                    in_s0, in_s1, out_s0, out_s1):
    core = lax.axis_index("c")
    sub = lax.axis_index("s")
    wid = core * SC_SUBCORES + sub
    pltpu.sync_copy(rcv_hbm.at[wid], idxr_v)
    pltpu.sync_copy(src_hbm.at[wid], idxs_v)
    pltpu.sync_copy(xpl_hbm, xpl_v)

    sets = ((buf_a0, buf_b0, buf_r0, in_s0, out_s0),
            (buf_a1, buf_b1, buf_r1, in_s1, out_s1))

    zero16 = jnp.zeros((16,), jnp.float32)

    def zrel(i, carry):
        for _, _, br_, _, _ in sets:
            br_[i, pl.ds(0, 16)] = zero16
        return carry

    lax.fori_loop(0, CHUNK, zrel, 0)

    iota = lax.broadcasted_iota(jnp.int32, (16,), 0)
    tail_mask = iota >= 8

    def issue(j, b):
        ba, bb, _, ins, _ = sets[b]
        pltpu.async_copy(a_hbm.at[idxr_v.at[j]], ba, ins)
        pltpu.async_copy(b_hbm.at[idxs_v.at[j]], bb, ins)

    def drain_in(b):
        ba, bb, _, ins, _ = sets[b]
        pltpu.make_async_copy(a_hbm.at[pl.ds(0, CHUNK)], ba, ins).wait()
        pltpu.make_async_copy(b_hbm.at[pl.ds(0, CHUNK)], bb, ins).wait()

    def rel_compute(j, b):
        _, _, br_, _, _ = sets[b]
        for off, mask in ((0, None), (16, None), (24, tail_mask)):
            ivr = idxr_v[j, pl.ds(off, 16)]
            ivs = idxs_v[j, pl.ds(off, 16)]
            rows = iota + off
            d2 = zero16
            for c in range(3):
                cc = jnp.full((16,), c, jnp.int32)
                xr = plsc.load_gather(xpl_v, [cc, ivr])
                xs = plsc.load_gather(xpl_v, [cc, ivs])
                rel = xr - xs
                d2 = d2 + rel * rel
                plsc.store_scatter(br_, [rows, cc], rel, mask=mask)
            plsc.store_scatter(br_, [rows, jnp.full((16,), 3, jnp.int32)],
                               d2, mask=mask)

    def out_copies(j, b):
        ba, bb, br_, _, outs = sets[b]
        base = wid * EPW + j * CHUNK
        pltpu.async_copy(ba, ar_out.at[pl.ds(base, CHUNK)], outs)
        pltpu.async_copy(bb, br_out.at[pl.ds(base, CHUNK)], outs)
        pltpu.async_copy(br_, xrel_out.at[pl.ds(base, CHUNK)], outs)

    def drain_out(b):
        ba, bb, br_, _, outs = sets[b]
        pltpu.make_async_copy(ba, ar_out.at[pl.ds(0, CHUNK)], outs).wait()
        pltpu.make_async_copy(bb, br_out.at[pl.ds(0, CHUNK)], outs).wait()
        pltpu.make_async_copy(br_, xrel_out.at[pl.ds(0, CHUNK)], outs).wait()

    issue(0, 0)
    issue(1, 1)

    def body(k, carry):
        for b in range(2):
            j = 2 * k + b
            rel_compute(j, b)
            drain_in(b)
            out_copies(j, b)
            drain_out(b)

            @pl.when(j + 2 < NCHUNK)
            def _():
                issue(j + 2, b)
        return carry

    lax.fori_loop(0, NCHUNK // 2, body, 0)
    for b in range(NCHUNK - 2 * (NCHUNK // 2)):
        j = 2 * (NCHUNK // 2) + b
        rel_compute(j, b)
        drain_in(b)
        out_copies(j, b)
        drain_out(b)


@functools.cache
def _sc_gather_fn():
    return pl.kernel(
        _sc_gather_body,
        out_type=[
            jax.ShapeDtypeStruct((N_EDGES, 256), jnp.uint32),
            jax.ShapeDtypeStruct((N_EDGES, 256), jnp.uint32),
            jax.ShapeDtypeStruct((N_EDGES, DIM_E), jnp.float32),
        ],
        mesh=_sc_mesh(),
        scratch_types=[
            pltpu.VMEM((NCHUNK, CHUNK), jnp.int32),
            pltpu.VMEM((NCHUNK, CHUNK), jnp.int32),
            pltpu.VMEM((3, N_NODES), jnp.float32),
            pltpu.VMEM((CHUNK, 256), jnp.uint32),
            pltpu.VMEM((CHUNK, 256), jnp.uint32),
            pltpu.VMEM((CHUNK, DIM_E), jnp.float32),
            pltpu.VMEM((CHUNK, 256), jnp.uint32),
            pltpu.VMEM((CHUNK, 256), jnp.uint32),
            pltpu.VMEM((CHUNK, DIM_E), jnp.float32),
            pltpu.SemaphoreType.DMA,
            pltpu.SemaphoreType.DMA,
            pltpu.SemaphoreType.DMA,
            pltpu.SemaphoreType.DMA,
        ],
        compiler_params=pltpu.CompilerParams(needs_layout_passes=False),
    )


def _sc_gather(*args):
    return _sc_gather_fn()(*args)


def _sc_scatter_m_body(m0, m1, m2, m3, rc, zeros_hbm, rcv_hbm,
                       g0, g1, g2, g3, gx,
                       idx_v, mbuf0, mbuf1, mbuf2, mbuf3, mbuf4, mbuf5, acc,
                       in_s0, in_s1, in_s2, in_s3, in_s4, in_s5,
                       add_s0, add_s1, add_s2, add_s3, add_s4, add_s5):
    core = lax.axis_index("c")
    sub = lax.axis_index("s")
    wid = core * SC_SUBCORES + sub
    pltpu.sync_copy(rcv_hbm.at[wid], idx_v)

    stripe0 = sub * NPW
    sets = ((mbuf0, in_s0, add_s0), (mbuf1, in_s1, add_s1),
            (mbuf2, in_s2, add_s2), (mbuf3, in_s3, add_s3),
            (mbuf4, in_s4, add_s4), (mbuf5, in_s5, add_s5))
    NS = 6

    for m_hbm, g_out in ((m0, g0), (m1, g1), (m2, g2), (m3, g3), (rc, gx)):
        for t in range(NPW // ZROWS):
            pltpu.sync_copy(zeros_hbm,
                            acc.at[pl.ds(stripe0 + t * ZROWS, ZROWS)])
        plsc.subcore_barrier()

        def issue_in(j, b, m_hbm=m_hbm):
            buf, ins, _ = sets[b]
            base = wid * EPW + j * CHUNK
            pltpu.async_copy(m_hbm.at[pl.ds(base, CHUNK)], buf, ins)

        def drain_in(b, m_hbm=m_hbm):
            buf, ins, _ = sets[b]
            pltpu.make_async_copy(m_hbm.at[pl.ds(0, CHUNK)], buf, ins).wait()

        def add_sync(j, b):
            buf, _, adds = sets[b]
            pltpu.async_copy(buf, acc.at[idx_v.at[j]], adds, add=True).wait()

        for b in range(NS):
            issue_in(b, b)

        def body(k, carry):
            for b in range(NS):
                j = NS * k + b
                drain_in(b)
                add_sync(j, b)

                @pl.when(j + NS < NCHUNK)
                def _():
                    issue_in(j + NS, b)
            return carry

        lax.fori_loop(0, NCHUNK // NS, body, 0)
        for b in range(NCHUNK - NS * (NCHUNK // NS)):
            j = NS * (NCHUNK // NS) + b
            drain_in(b)
            add_sync(j, b)
        plsc.subcore_barrier()
        pltpu.sync_copy(acc.at[pl.ds(stripe0, NPW)],
                        g_out.at[core, pl.ds(stripe0, NPW)])
        plsc.subcore_barrier()


@functools.cache
def _sc_scatter_m_fn():
    return pl.kernel(
        _sc_scatter_m_body,
        out_type=[
            jax.ShapeDtypeStruct((SC_CORES, N_PAD, 128), jnp.float32),
            jax.ShapeDtypeStruct((SC_CORES, N_PAD, 128), jnp.float32),
            jax.ShapeDtypeStruct((SC_CORES, N_PAD, 128), jnp.float32),
            jax.ShapeDtypeStruct((SC_CORES, N_PAD, 128), jnp.float32),
            jax.ShapeDtypeStruct((SC_CORES, N_PAD, 128), jnp.float32),
        ],
        mesh=_sc_mesh(),
        scratch_types=[
            pltpu.VMEM((NCHUNK, CHUNK), jnp.int32),
            pltpu.VMEM((CHUNK, 128), jnp.float32),
            pltpu.VMEM((CHUNK, 128), jnp.float32),
            pltpu.VMEM((CHUNK, 128), jnp.float32),
            pltpu.VMEM((CHUNK, 128), jnp.float32),
            pltpu.VMEM((CHUNK, 128), jnp.float32),
            pltpu.VMEM((CHUNK, 128), jnp.float32),
            pltpu.VMEM_SHARED((N_PAD, 128), jnp.float32),
        ] + [pltpu.SemaphoreType.DMA] * 12,
    )


def _sc_scatter(m0, m1, m2, m3, rc, rcv2):
    zeros = jnp.zeros((ZROWS, 128), jnp.float32)
    return _sc_scatter_m_fn()(m0, m1, m2, m3, rc, zeros, rcv2)


def _mm(a, b):
    return jnp.dot(a, b, preferred_element_type=jnp.float32)


def _pack_half(x):
    lo = lax.bitcast_convert_type(
        x[:, :DIM_H // 2].astype(jnp.bfloat16), jnp.uint16).astype(jnp.uint32)
    hi = lax.bitcast_convert_type(
        x[:, DIM_H // 2:].astype(jnp.bfloat16), jnp.uint16).astype(jnp.uint32)
    return lo | (hi << 16)


def _unpack_half(w):
    lo = lax.bitcast_convert_type(w.astype(jnp.uint16), jnp.bfloat16)
    hi = lax.bitcast_convert_type((w >> 16).astype(jnp.uint16), jnp.bfloat16)
    return lo.astype(jnp.float32), hi.astype(jnp.float32)


def _prologue_body(feat, w_in, b_in, we1i, we1j, h0_o, a_o, b_o):
    h0 = _mm(feat[...], w_in[...]) + b_in[...]
    h0_o[...] = h0
    a_o[...] = _pack_half(_mm(h0, we1i[...]))
    b_o[...] = _pack_half(_mm(h0, we1j[...]))


def _tc_prologue(feat, w_in, b_in, we1i, we1j):
    grid = (N_NODES // BN,)
    row = pl.BlockSpec((BN, DIM_IN), lambda i: (i, 0))
    out = pl.BlockSpec((BN, DIM_H), lambda i: (i, 0))
    outT = pl.BlockSpec((BN, DIM_H // 2), lambda i: (i, 0))
    full = lambda shape: pl.BlockSpec(shape, lambda i: (0,) * len(shape))
    return pl.pallas_call(
        _prologue_body,
        grid=grid,
        in_specs=[row, full((DIM_IN, DIM_H)), full((1, DIM_H)),
                  full((DIM_H, DIM_H)), full((DIM_H, DIM_H))],
        out_specs=[out, outT, outT],
        out_shape=[jax.ShapeDtypeStruct((N_NODES, DIM_H), jnp.float32),
                   jax.ShapeDtypeStruct((N_NODES, DIM_H // 2), jnp.uint32),
                   jax.ShapeDtypeStruct((N_NODES, DIM_H // 2), jnp.uint32)],
    )(feat, w_in, b_in, we1i, we1j)


def _edge_body(ar, br, xrel, ea, we1e, wd, be1, we2, be2, wx, bx,
               m0_o, m1_o, m2_o, m3_o, rc_o):
    alo, ahi = _unpack_half(ar[...])
    blo, bhi = _unpack_half(br[...])
    pre = jnp.concatenate([alo + blo, ahi + bhi], axis=1)
    xre = xrel[...]
    d2 = xre[:, 3:4]
    t = pre + d2 * wd[...] + _mm(ea[...], we1e[...]) + be1[...]
    t = jax.nn.silu(t).astype(jnp.bfloat16)
    m = jax.nn.silu(_mm(t, we2[...].astype(jnp.bfloat16)) + be2[...])
    cw = jnp.tanh(jnp.sum(m * wx[...], axis=1, keepdims=True) + bx[...])
    col = lax.broadcasted_iota(jnp.int32, (BE, DIM_E), 1)
    rel = jnp.where(col < 3, xre, 0.0)
    rc16 = rel * cw + jnp.where(col == 3, 1.0, 0.0)
    rc = jnp.concatenate([rc16, jnp.zeros((BE, XW - DIM_E), jnp.float32)],
                         axis=1)
    m0_o[...] = m[:, 0:128]
    m1_o[...] = m[:, 128:256]
    m2_o[...] = m[:, 256:384]
    m3_o[...] = m[:, 384:512]
    rc_o[...] = rc


def _tc_edge(ar, br, xrel, ea, we1e, wd, be1, we2, be2, wx, bx):
    grid = (N_EDGES // BE,)
    rowH = pl.BlockSpec((BE, DIM_H // 2), lambda i: (i, 0))
    rowE = pl.BlockSpec((BE, DIM_E), lambda i: (i, 0))
    rowX = pl.BlockSpec((BE, XW), lambda i: (i, 0))
    full = lambda shape: pl.BlockSpec(shape, lambda i: (0,) * len(shape))
    out128 = pl.BlockSpec((BE, 128), lambda i: (i, 0))
    return pl.pallas_call(
        _edge_body,
        grid=grid,
        in_specs=[rowH, rowH, rowE, rowE,
                  full((DIM_E, DIM_H)), full((1, DIM_H)), full((1, DIM_H)),
                  full((DIM_H, DIM_H)), full((1, DIM_H)), full((1, DIM_H)),
                  full((1, 1))],
        out_specs=[out128, out128, out128, out128, out128],
        out_shape=[jax.ShapeDtypeStruct((N_EDGES, 128), jnp.float32)] * 5,
    )(ar, br, xrel, ea, we1e, wd, be1, we2, be2, wx, bx)


def _node_common(h_ref, g0, g1, g2, g3, gx, wh1h, wh1a, bh1, wh2, bh2):
    h = h_ref[...]
    agg = jnp.concatenate(
        [g0[0] + g0[1], g1[0] + g1[1], g2[0] + g2[1], g3[0] + g3[1]], axis=1)
    xa = gx[0] + gx[1]
    deg = xa[:, 3:4]

    invd = 1.0 / jnp.maximum(deg, 1.0)
    agg = agg * invd
    u = jax.nn.silu(_mm(h, wh1h[...]) + _mm(agg, wh1a[...]) + bh1[...])
    h_new = h + _mm(u, wh2[...]) + bh2[...]
    return h_new, xa, invd


def _node_body(h_ref, xp, g0, g1, g2, g3, gx,
               wh1h, wh1a, bh1, wh2, bh2, we1i, we1j,
               h_o, x_o, a_o, b_o):
    h_new, xa, invd = _node_common(h_ref, g0, g1, g2, g3, gx,
                                   wh1h, wh1a, bh1, wh2, bh2)
    col = lax.broadcasted_iota(jnp.int32, (BN, XW), 1)
    mask3 = jnp.where(col < 3, 1.0, 0.0)
    x_o[...] = xp[...] + xa * invd * mask3
    h_o[...] = h_new
    a_o[...] = _pack_half(_mm(h_new, we1i[...]))
    b_o[...] = _pack_half(_mm(h_new, we1j[...]))


def _final_body(h_ref, xp, g0, g1, g2, g3, gx,
                wh1h, wh1a, bh1, wh2, bh2, h_o, gp_o):
    h_new, _, _ = _node_common(h_ref, g0, g1, g2, g3, gx,
                               wh1h, wh1a, bh1, wh2, bh2)
    h_o[...] = h_new

    @pl.when(pl.program_id(0) == 0)
    def _():
        gp_o[...] = jnp.zeros_like(gp_o)

    gp_o[...] += jnp.sum(h_new, axis=0, keepdims=True) / N_NODES


def _node_specs():
    rowH = pl.BlockSpec((BN, DIM_H), lambda i: (i, 0))
    rowE = pl.BlockSpec((BN, XW), lambda i: (i, 0))
    g128 = pl.BlockSpec((SC_CORES, BN, 128), lambda i: (0, i, 0))
    gE = g128
    full = lambda shape: pl.BlockSpec(shape, lambda i: (0,) * len(shape))
    w = full((DIM_H, DIM_H))
    b = full((1, DIM_H))
    return rowH, rowE, g128, gE, w, b


def _tc_node(h, xp, gs, gx, wh1h, wh1a, bh1, wh2, bh2, we1i, we1j):
    rowH, rowE, g128, gE, w, b = _node_specs()
    outT = pl.BlockSpec((BN, DIM_H // 2), lambda i: (i, 0))
    return pl.pallas_call(
        _node_body,
        grid=(N_NODES // BN,),
        in_specs=[rowH, rowE, g128, g128, g128, g128, gE, w, w, b, w, b, w, w],
        out_specs=[rowH, rowE, outT, outT],
        out_shape=[
            jax.ShapeDtypeStruct((N_NODES, DIM_H), jnp.float32),
            jax.ShapeDtypeStruct((N_NODES, XW), jnp.float32),
            jax.ShapeDtypeStruct((N_NODES, DIM_H // 2), jnp.uint32),
            jax.ShapeDtypeStruct((N_NODES, DIM_H // 2), jnp.uint32),
        ],
    )(h, xp, *gs, gx, wh1h, wh1a, bh1, wh2, bh2, we1i, we1j)


def _tc_final(h, xp, gs, gx, wh1h, wh1a, bh1, wh2, bh2):
    rowH, rowE, g128, gE, w, b = _node_specs()
    gp = pl.BlockSpec((1, DIM_H), lambda i: (0, 0))
    return pl.pallas_call(
        _final_body,
        grid=(N_NODES // BN,),
        in_specs=[rowH, rowE, g128, g128, g128, g128, gE, w, w, b, w, b],
        out_specs=[rowH, gp],
        out_shape=[
            jax.ShapeDtypeStruct((N_NODES, DIM_H), jnp.float32),
            jax.ShapeDtypeStruct((1, DIM_H), jnp.float32),
        ],
    )(h, xp, *gs, gx, wh1h, wh1a, bh1, wh2, bh2)


def kernel(protein_pos, protein_atom_feature, pp_edge_index, pp_edge_attr,
           params):
    src = pp_edge_index[0].astype(jnp.int32)
    rcv = pp_edge_index[1].astype(jnp.int32)
    rcv2 = rcv.reshape(NW, NCHUNK, CHUNK)
    src2 = src.reshape(NW, NCHUNK, CHUNK)
    xp = jnp.pad(protein_pos.astype(jnp.float32), ((0, 0), (0, XW - 3)))

    layers = params['layers']

    def wsplit(p):
        we1 = p['We1']
        return (we1[0:DIM_H], we1[DIM_H:2 * DIM_H],
                we1[2 * DIM_H:2 * DIM_H + 1],
                we1[2 * DIM_H + 1:])

    we1i0, we1j0, _, _ = wsplit(layers[0])
    h, a, b = _tc_prologue(
        protein_atom_feature, params['W_in'], params['b_in'].reshape(1, DIM_H),
        we1i0, we1j0)

    for l in range(N_LAYERS):
        p = layers[l]
        _, _, wd, we1e = wsplit(p)
        xpl = jnp.transpose(xp[:, :3])
        ar, br, xrel = _sc_gather(a, b, xpl, rcv2, src2)
        m0, m1, m2, m3, rc = _tc_edge(
            ar, br, xrel, pp_edge_attr,
            we1e, wd, p['be1'].reshape(1, DIM_H),
            p['We2'], p['be2'].reshape(1, DIM_H),
            p['Wx'].reshape(1, DIM_H), p['bx'].reshape(1, 1))
        g0, g1, g2, g3, gx = _sc_scatter(m0, m1, m2, m3, rc, rcv2)
        wh1 = p['Wh1']
        if l < N_LAYERS - 1:
            we1i_n, we1j_n, _, _ = wsplit(layers[l + 1])
            h, xp, a, b = _tc_node(
                h, xp, (g0, g1, g2, g3), gx,
                wh1[0:DIM_H], wh1[DIM_H:], p['bh1'].reshape(1, DIM_H),
                p['Wh2'], p['bh2'].reshape(1, DIM_H), we1i_n, we1j_n)
        else:
            h, gp = _tc_final(
                h, xp, (g0, g1, g2, g3), gx,
                wh1[0:DIM_H], wh1[DIM_H:], p['bh1'].reshape(1, DIM_H),
                p['Wh2'], p['bh2'].reshape(1, DIM_H))

    return (h, gp.reshape(DIM_H))


# speedup vs baseline: 1.1063x; 1.0134x over previous
import functools

import jax
import jax.numpy as jnp
from jax import lax
from jax.experimental import pallas as pl
from jax.experimental.pallas import tpu as pltpu
from jax.experimental.pallas import tpu_sc as plsc

N_NODES = 10000
N_EDGES = 160000
DIM_IN = 256
DIM_H = 512
DIM_E = 16
N_LAYERS = 4

SC_CORES = 2
SC_SUBCORES = 16
NW = SC_CORES * SC_SUBCORES
EPW = N_EDGES // NW
CHUNK = 40
NCHUNK = EPW // CHUNK
N_PAD = 10240
NPW = N_PAD // SC_SUBCORES
ZROWS = 128

XW = 128
BN = 1000
BE = 1000

@functools.cache
def _sc_mesh():
    return plsc.VectorSubcoreMesh(core_axis_name="c", subcore_axis_name="s",
                                  num_cores=SC_CORES,
                                  num_subcores=SC_SUBCORES)


def _sc_gather_body(a_hbm, b_hbm, xpl_hbm, rcv_hbm, src_hbm,
                    ar_out, br_out, xrel_out,
                    idxr_v, idxs_v, xpl_v,
                    buf_a0, buf_b0, buf_r0,
                    buf_a1, buf_b1, buf_r1,
                    in_s0, in_s1, out_s0, out_s1):
    core = lax.axis_index("c")
    sub = lax.axis_index("s")
    wid = core * SC_SUBCORES + sub
    pltpu.sync_copy(rcv_hbm.at[wid], idxr_v)
    pltpu.sync_copy(src_hbm.at[wid], idxs_v)
    pltpu.sync_copy(xpl_hbm, xpl_v)

    sets = ((buf_a0, buf_b0, buf_r0, in_s0, out_s0),
            (buf_a1, buf_b1, buf_r1, in_s1, out_s1))

    zero16 = jnp.zeros((16,), jnp.float32)

    def zrel(i, carry):
        for _, _, br_, _, _ in sets:
            br_[i, pl.ds(0, 16)] = zero16
        return carry

    lax.fori_loop(0, CHUNK, zrel, 0)

    iota = lax.broadcasted_iota(jnp.int32, (16,), 0)
    tail_mask = iota >= 8

    def issue(j, b):
        ba, bb, _, ins, _ = sets[b]
        pltpu.async_copy(a_hbm.at[idxr_v.at[j]], ba, ins)
        pltpu.async_copy(b_hbm.at[idxs_v.at[j]], bb, ins)

    def drain_in(b):
        ba, bb, _, ins, _ = sets[b]
        pltpu.make_async_copy(a_hbm.at[pl.ds(0, CHUNK)], ba, ins).wait()
        pltpu.make_async_copy(b_hbm.at[pl.ds(0, CHUNK)], bb, ins).wait()

    def rel_compute(j, b):
        _, _, br_, _, _ = sets[b]
        for off, mask in ((0, None), (16, None), (24, tail_mask)):
            ivr = idxr_v[j, pl.ds(off, 16)]
            ivs = idxs_v[j, pl.ds(off, 16)]
            rows = iota + off
            d2 = zero16
            for c in range(3):
                cc = jnp.full((16,), c, jnp.int32)
                xr = plsc.load_gather(xpl_v, [cc, ivr])
                xs = plsc.load_gather(xpl_v, [cc, ivs])
                rel = xr - xs
                d2 = d2 + rel * rel
                plsc.store_scatter(br_, [rows, cc], rel, mask=mask)
            plsc.store_scatter(br_, [rows, jnp.full((16,), 3, jnp.int32)],
                               d2, mask=mask)

    def out_copies(j, b):
        ba, bb, br_, _, outs = sets[b]
        base = wid * EPW + j * CHUNK
        pltpu.async_copy(ba, ar_out.at[pl.ds(base, CHUNK)], outs)
        pltpu.async_copy(bb, br_out.at[pl.ds(base, CHUNK)], outs)
        pltpu.async_copy(br_, xrel_out.at[pl.ds(base, CHUNK)], outs)

    def drain_out(b):
        ba, bb, br_, _, outs = sets[b]
        pltpu.make_async_copy(ba, ar_out.at[pl.ds(0, CHUNK)], outs).wait()
        pltpu.make_async_copy(bb, br_out.at[pl.ds(0, CHUNK)], outs).wait()
        pltpu.make_async_copy(br_, xrel_out.at[pl.ds(0, CHUNK)], outs).wait()

    issue(0, 0)
    issue(1, 1)

    def body(k, carry):
        for b in range(2):
            j = 2 * k + b
            rel_compute(j, b)
            drain_in(b)
            out_copies(j, b)
            drain_out(b)

            @pl.when(j + 2 < NCHUNK)
            def _():
                issue(j + 2, b)
        return carry

    lax.fori_loop(0, NCHUNK // 2, body, 0)
    for b in range(NCHUNK - 2 * (NCHUNK // 2)):
        j = 2 * (NCHUNK // 2) + b
        rel_compute(j, b)
        drain_in(b)
        out_copies(j, b)
        drain_out(b)


@functools.cache
def _sc_gather_fn():
    return pl.kernel(
        _sc_gather_body,
        out_type=[
            jax.ShapeDtypeStruct((N_EDGES, 256), jnp.uint32),
            jax.ShapeDtypeStruct((N_EDGES, 256), jnp.uint32),
            jax.ShapeDtypeStruct((N_EDGES, DIM_E), jnp.float32),
        ],
        mesh=_sc_mesh(),
        scratch_types=[
            pltpu.VMEM((NCHUNK, CHUNK), jnp.int32),
            pltpu.VMEM((NCHUNK, CHUNK), jnp.int32),
            pltpu.VMEM((3, N_NODES), jnp.float32),
            pltpu.VMEM((CHUNK, 256), jnp.uint32),
            pltpu.VMEM((CHUNK, 256), jnp.uint32),
            pltpu.VMEM((CHUNK, DIM_E), jnp.float32),
            pltpu.VMEM((CHUNK, 256), jnp.uint32),
            pltpu.VMEM((CHUNK, 256), jnp.uint32),
            pltpu.VMEM((CHUNK, DIM_E), jnp.float32),
            pltpu.SemaphoreType.DMA,
            pltpu.SemaphoreType.DMA,
            pltpu.SemaphoreType.DMA,
            pltpu.SemaphoreType.DMA,
        ],
        compiler_params=pltpu.CompilerParams(needs_layout_passes=False),
    )


def _sc_gather(*args):
    return _sc_gather_fn()(*args)


def _sc_scatter_m_body(m0, m1, m2, m3, rc, zeros_hbm, rcv_hbm,
                       g0, g1, g2, g3, gx,
                       idx_v, mbuf0, mbuf1, mbuf2, mbuf3, mbuf4, mbuf5, acc,
                       in_s0, in_s1, in_s2, in_s3, in_s4, in_s5,
                       add_s0, add_s1, add_s2, add_s3, add_s4, add_s5):
    core = lax.axis_index("c")
    sub = lax.axis_index("s")
    wid = core * SC_SUBCORES + sub
    pltpu.sync_copy(rcv_hbm.at[wid], idx_v)

    stripe0 = sub * NPW
    sets = ((mbuf0, in_s0, add_s0), (mbuf1, in_s1, add_s1),
            (mbuf2, in_s2, add_s2), (mbuf3, in_s3, add_s3),
            (mbuf4, in_s4, add_s4), (mbuf5, in_s5, add_s5))
    NS = 6

    for m_hbm, g_out in ((m0, g0), (m1, g1), (m2, g2), (m3, g3), (rc, gx)):
        for t in range(NPW // ZROWS):
            pltpu.sync_copy(zeros_hbm,
                            acc.at[pl.ds(stripe0 + t * ZROWS, ZROWS)])
        plsc.subcore_barrier()

        def issue_in(j, b, m_hbm=m_hbm):
            buf, ins, _ = sets[b]
            base = wid * EPW + j * CHUNK
            pltpu.async_copy(m_hbm.at[pl.ds(base, CHUNK)], buf, ins)

        def drain_in(b, m_hbm=m_hbm):
            buf, ins, _ = sets[b]
            pltpu.make_async_copy(m_hbm.at[pl.ds(0, CHUNK)], buf, ins).wait()

        def add_sync(j, b):
            buf, _, adds = sets[b]
            pltpu.async_copy(buf, acc.at[idx_v.at[j]], adds, add=True).wait()

        for b in range(NS):
            issue_in(b, b)

        def body(k, carry):
            for b in range(NS):
                j = NS * k + b
                drain_in(b)
                add_sync(j, b)

                @pl.when(j + NS < NCHUNK)
                def _():
                    issue_in(j + NS, b)
            return carry

        lax.fori_loop(0, NCHUNK // NS, body, 0)
        for b in range(NCHUNK - NS * (NCHUNK // NS)):
            j = NS * (NCHUNK // NS) + b
            drain_in(b)
            add_sync(j, b)
        plsc.subcore_barrier()
        pltpu.sync_copy(acc.at[pl.ds(stripe0, NPW)],
                        g_out.at[core, pl.ds(stripe0, NPW)])
        plsc.subcore_barrier()


@functools.cache
def _sc_scatter_m_fn():
    return pl.kernel(
        _sc_scatter_m_body,
        out_type=[
            jax.ShapeDtypeStruct((SC_CORES, N_PAD, 128), jnp.float32),
            jax.ShapeDtypeStruct((SC_CORES, N_PAD, 128), jnp.float32),
            jax.ShapeDtypeStruct((SC_CORES, N_PAD, 128), jnp.float32),
            jax.ShapeDtypeStruct((SC_CORES, N_PAD, 128), jnp.float32),
            jax.ShapeDtypeStruct((SC_CORES, N_PAD, 128), jnp.float32),
        ],
        mesh=_sc_mesh(),
        scratch_types=[
            pltpu.VMEM((NCHUNK, CHUNK), jnp.int32),
            pltpu.VMEM((CHUNK, 128), jnp.float32),
            pltpu.VMEM((CHUNK, 128), jnp.float32),
            pltpu.VMEM((CHUNK, 128), jnp.float32),
            pltpu.VMEM((CHUNK, 128), jnp.float32),
            pltpu.VMEM((CHUNK, 128), jnp.float32),
            pltpu.VMEM((CHUNK, 128), jnp.float32),
            pltpu.VMEM_SHARED((N_PAD, 128), jnp.float32),
        ] + [pltpu.SemaphoreType.DMA] * 12,
    )


def _sc_scatter4_body(m0, m1, m2, m3, zeros_hbm, rcv_hbm,
                      g0, g1, g2, g3,
                      idx_v, mbuf0, mbuf1, mbuf2, mbuf3, mbuf4, mbuf5, acc,
                      in_s0, in_s1, in_s2, in_s3, in_s4, in_s5,
                      add_s0, add_s1, add_s2, add_s3, add_s4, add_s5):
    core = lax.axis_index("c")
    sub = lax.axis_index("s")
    wid = core * SC_SUBCORES + sub
    pltpu.sync_copy(rcv_hbm.at[wid], idx_v)

    stripe0 = sub * NPW
    sets = ((mbuf0, in_s0, add_s0), (mbuf1, in_s1, add_s1),
            (mbuf2, in_s2, add_s2), (mbuf3, in_s3, add_s3),
            (mbuf4, in_s4, add_s4), (mbuf5, in_s5, add_s5))
    NS = 6

    for m_hbm, g_out in ((m0, g0), (m1, g1), (m2, g2), (m3, g3)):
        for t in range(NPW // ZROWS):
            pltpu.sync_copy(zeros_hbm,
                            acc.at[pl.ds(stripe0 + t * ZROWS, ZROWS)])
        plsc.subcore_barrier()

        def issue_in(j, b, m_hbm=m_hbm):
            buf, ins, _ = sets[b]
            base = wid * EPW + j * CHUNK
            pltpu.async_copy(m_hbm.at[pl.ds(base, CHUNK)], buf, ins)

        def drain_in(b, m_hbm=m_hbm):
            buf, ins, _ = sets[b]
            pltpu.make_async_copy(m_hbm.at[pl.ds(0, CHUNK)], buf, ins).wait()

        def add_sync(j, b):
            buf, _, adds = sets[b]
            pltpu.async_copy(buf, acc.at[idx_v.at[j]], adds, add=True).wait()

        for b in range(NS):
            issue_in(b, b)

        def body(k, carry):
            for b in range(NS):
                j = NS * k + b
                drain_in(b)
                add_sync(j, b)

                @pl.when(j + NS < NCHUNK)
                def _():
                    issue_in(j + NS, b)
            return carry

        lax.fori_loop(0, NCHUNK // NS, body, 0)
        for b in range(NCHUNK - NS * (NCHUNK // NS)):
            j = NS * (NCHUNK // NS) + b
            drain_in(b)
            add_sync(j, b)
        plsc.subcore_barrier()
        pltpu.sync_copy(acc.at[pl.ds(stripe0, NPW)],
                        g_out.at[core, pl.ds(stripe0, NPW)])
        plsc.subcore_barrier()


@functools.cache
def _sc_scatter4_fn():
    return pl.kernel(
        _sc_scatter4_body,
        out_type=[
            jax.ShapeDtypeStruct((SC_CORES, N_PAD, 128), jnp.float32),
            jax.ShapeDtypeStruct((SC_CORES, N_PAD, 128), jnp.float32),
            jax.ShapeDtypeStruct((SC_CORES, N_PAD, 128), jnp.float32),
            jax.ShapeDtypeStruct((SC_CORES, N_PAD, 128), jnp.float32),
        ],
        mesh=_sc_mesh(),
        scratch_types=[
            pltpu.VMEM((NCHUNK, CHUNK), jnp.int32),
            pltpu.VMEM((CHUNK, 128), jnp.float32),
            pltpu.VMEM((CHUNK, 128), jnp.float32),
            pltpu.VMEM((CHUNK, 128), jnp.float32),
            pltpu.VMEM((CHUNK, 128), jnp.float32),
            pltpu.VMEM((CHUNK, 128), jnp.float32),
            pltpu.VMEM((CHUNK, 128), jnp.float32),
            pltpu.VMEM_SHARED((N_PAD, 128), jnp.float32),
        ] + [pltpu.SemaphoreType.DMA] * 12,
    )


def _sc_scatter4(m0, m1, m2, m3, rcv2):
    zeros = jnp.zeros((ZROWS, 128), jnp.float32)
    return _sc_scatter4_fn()(m0, m1, m2, m3, zeros, rcv2)


def _sc_scatter(m0, m1, m2, m3, rc, rcv2):
    zeros = jnp.zeros((ZROWS, 128), jnp.float32)
    return _sc_scatter_m_fn()(m0, m1, m2, m3, rc, zeros, rcv2)


def _mm(a, b):
    return jnp.dot(a, b, preferred_element_type=jnp.float32)


def _pack_half(x):
    lo = lax.bitcast_convert_type(
        x[:, :DIM_H // 2].astype(jnp.bfloat16), jnp.uint16).astype(jnp.uint32)
    hi = lax.bitcast_convert_type(
        x[:, DIM_H // 2:].astype(jnp.bfloat16), jnp.uint16).astype(jnp.uint32)
    return lo | (hi << 16)


def _unpack_half(w):
    lo = lax.bitcast_convert_type(w.astype(jnp.uint16), jnp.bfloat16)
    hi = lax.bitcast_convert_type((w >> 16).astype(jnp.uint16), jnp.bfloat16)
    return lo.astype(jnp.float32), hi.astype(jnp.float32)


def _prologue_body(feat, w_in, b_in, we1i, we1j, h0_o, a_o, b_o):
    h0 = _mm(feat[...], w_in[...]) + b_in[...]
    h0_o[...] = h0
    a_o[...] = _pack_half(_mm(h0, we1i[...]))
    b_o[...] = _pack_half(_mm(h0, we1j[...]))


def _tc_prologue(feat, w_in, b_in, we1i, we1j):
    grid = (N_NODES // BN,)
    row = pl.BlockSpec((BN, DIM_IN), lambda i: (i, 0))
    out = pl.BlockSpec((BN, DIM_H), lambda i: (i, 0))
    outT = pl.BlockSpec((BN, DIM_H // 2), lambda i: (i, 0))
    full = lambda shape: pl.BlockSpec(shape, lambda i: (0,) * len(shape))
    return pl.pallas_call(
        _prologue_body,
        grid=grid,
        in_specs=[row, full((DIM_IN, DIM_H)), full((1, DIM_H)),
                  full((DIM_H, DIM_H)), full((DIM_H, DIM_H))],
        out_specs=[out, outT, outT],
        out_shape=[jax.ShapeDtypeStruct((N_NODES, DIM_H), jnp.float32),
                   jax.ShapeDtypeStruct((N_NODES, DIM_H // 2), jnp.uint32),
                   jax.ShapeDtypeStruct((N_NODES, DIM_H // 2), jnp.uint32)],
    )(feat, w_in, b_in, we1i, we1j)


def _edge_body(ar, br, xrel, ea, we1e, wd, be1, we2, be2, wx, bx,
               m0_o, m1_o, m2_o, m3_o, rc_o):
    alo, ahi = _unpack_half(ar[...])
    blo, bhi = _unpack_half(br[...])
    pre = jnp.concatenate([alo + blo, ahi + bhi], axis=1)
    xre = xrel[...]
    d2 = xre[:, 3:4]
    t = pre + d2 * wd[...] + _mm(ea[...], we1e[...]) + be1[...]
    t = jax.nn.silu(t).astype(jnp.bfloat16)
    m = jax.nn.silu(_mm(t, we2[...].astype(jnp.bfloat16)) + be2[...])
    cw = jnp.tanh(jnp.sum(m * wx[...], axis=1, keepdims=True) + bx[...])
    col = lax.broadcasted_iota(jnp.int32, (BE, DIM_E), 1)
    rel = jnp.where(col < 3, xre, 0.0)
    rc16 = rel * cw + jnp.where(col == 3, 1.0, 0.0)
    rc = jnp.concatenate([rc16, jnp.zeros((BE, XW - DIM_E), jnp.float32)],
                         axis=1)
    m0_o[...] = m[:, 0:128]
    m1_o[...] = m[:, 128:256]
    m2_o[...] = m[:, 256:384]
    m3_o[...] = m[:, 384:512]
    rc_o[...] = rc


def _tc_edge(ar, br, xrel, ea, we1e, wd, be1, we2, be2, wx, bx):
    grid = (N_EDGES // BE,)
    rowH = pl.BlockSpec((BE, DIM_H // 2), lambda i: (i, 0))
    rowE = pl.BlockSpec((BE, DIM_E), lambda i: (i, 0))
    rowX = pl.BlockSpec((BE, XW), lambda i: (i, 0))
    full = lambda shape: pl.BlockSpec(shape, lambda i: (0,) * len(shape))
    out128 = pl.BlockSpec((BE, 128), lambda i: (i, 0))
    return pl.pallas_call(
        _edge_body,
        grid=grid,
        in_specs=[rowH, rowH, rowE, rowE,
                  full((DIM_E, DIM_H)), full((1, DIM_H)), full((1, DIM_H)),
                  full((DIM_H, DIM_H)), full((1, DIM_H)), full((1, DIM_H)),
                  full((1, 1))],
        out_specs=[out128, out128, out128, out128, out128],
        out_shape=[jax.ShapeDtypeStruct((N_EDGES, 128), jnp.float32)] * 5,
    )(ar, br, xrel, ea, we1e, wd, be1, we2, be2, wx, bx)


def _node_common(h_ref, g0, g1, g2, g3, gx, wh1h, wh1a, bh1, wh2, bh2):
    h = h_ref[...]
    agg = jnp.concatenate(
        [g0[0] + g0[1], g1[0] + g1[1], g2[0] + g2[1], g3[0] + g3[1]], axis=1)
    xa = gx[0] + gx[1]
    deg = xa[:, 3:4]

    invd = 1.0 / jnp.maximum(deg, 1.0)
    agg = agg * invd
    u = jax.nn.silu(_mm(h, wh1h[...]) + _mm(agg, wh1a[...]) + bh1[...])
    h_new = h + _mm(u, wh2[...]) + bh2[...]
    return h_new, xa, invd


def _node_body(h_ref, xp, g0, g1, g2, g3, gx,
               wh1h, wh1a, bh1, wh2, bh2, we1i, we1j,
               h_o, x_o, a_o, b_o):
    h_new, xa, invd = _node_common(h_ref, g0, g1, g2, g3, gx,
                                   wh1h, wh1a, bh1, wh2, bh2)
    col = lax.broadcasted_iota(jnp.int32, (BN, XW), 1)
    mask3 = jnp.where(col < 3, 1.0, 0.0)
    x_o[...] = xp[...] + xa * invd * mask3
    h_o[...] = h_new
    a_o[...] = _pack_half(_mm(h_new, we1i[...]))
    b_o[...] = _pack_half(_mm(h_new, we1j[...]))


def _final_body(h_ref, xp, g0, g1, g2, g3, gx,
                wh1h, wh1a, bh1, wh2, bh2, h_o, gp_o):
    h_new, _, _ = _node_common(h_ref, g0, g1, g2, g3, gx,
                               wh1h, wh1a, bh1, wh2, bh2)
    h_o[...] = h_new

    @pl.when(pl.program_id(0) == 0)
    def _():
        gp_o[...] = jnp.zeros_like(gp_o)

    gp_o[...] += jnp.sum(h_new, axis=0, keepdims=True) / N_NODES


def _node_specs():
    rowH = pl.BlockSpec((BN, DIM_H), lambda i: (i, 0))
    rowE = pl.BlockSpec((BN, XW), lambda i: (i, 0))
    g128 = pl.BlockSpec((SC_CORES, BN, 128), lambda i: (0, i, 0))
    gE = g128
    full = lambda shape: pl.BlockSpec(shape, lambda i: (0,) * len(shape))
    w = full((DIM_H, DIM_H))
    b = full((1, DIM_H))
    return rowH, rowE, g128, gE, w, b


def _tc_node(h, xp, gs, gx, wh1h, wh1a, bh1, wh2, bh2, we1i, we1j):
    rowH, rowE, g128, gE, w, b = _node_specs()
    outT = pl.BlockSpec((BN, DIM_H // 2), lambda i: (i, 0))
    return pl.pallas_call(
        _node_body,
        grid=(N_NODES // BN,),
        in_specs=[rowH, rowE, g128, g128, g128, g128, gE, w, w, b, w, b, w, w],
        out_specs=[rowH, rowE, outT, outT],
        out_shape=[
            jax.ShapeDtypeStruct((N_NODES, DIM_H), jnp.float32),
            jax.ShapeDtypeStruct((N_NODES, XW), jnp.float32),
            jax.ShapeDtypeStruct((N_NODES, DIM_H // 2), jnp.uint32),
            jax.ShapeDtypeStruct((N_NODES, DIM_H // 2), jnp.uint32),
        ],
    )(h, xp, *gs, gx, wh1h, wh1a, bh1, wh2, bh2, we1i, we1j)


def _tc_final(h, xp, gs, gx, wh1h, wh1a, bh1, wh2, bh2):
    rowH, rowE, g128, gE, w, b = _node_specs()
    gp = pl.BlockSpec((1, DIM_H), lambda i: (0, 0))
    return pl.pallas_call(
        _final_body,
        grid=(N_NODES // BN,),
        in_specs=[rowH, rowE, g128, g128, g128, g128, gE, w, w, b, w, b],
        out_specs=[rowH, gp],
        out_shape=[
            jax.ShapeDtypeStruct((N_NODES, DIM_H), jnp.float32),
            jax.ShapeDtypeStruct((1, DIM_H), jnp.float32),
        ],
    )(h, xp, *gs, gx, wh1h, wh1a, bh1, wh2, bh2)


def kernel(protein_pos, protein_atom_feature, pp_edge_index, pp_edge_attr,
           params):
    src = pp_edge_index[0].astype(jnp.int32)
    rcv = pp_edge_index[1].astype(jnp.int32)
    rcv2 = rcv.reshape(NW, NCHUNK, CHUNK)
    src2 = src.reshape(NW, NCHUNK, CHUNK)
    xp = jnp.pad(protein_pos.astype(jnp.float32), ((0, 0), (0, XW - 3)))

    layers = params['layers']

    def wsplit(p):
        we1 = p['We1']
        return (we1[0:DIM_H], we1[DIM_H:2 * DIM_H],
                we1[2 * DIM_H:2 * DIM_H + 1],
                we1[2 * DIM_H + 1:])

    we1i0, we1j0, _, _ = wsplit(layers[0])
    h, a, b = _tc_prologue(
        protein_atom_feature, params['W_in'], params['b_in'].reshape(1, DIM_H),
        we1i0, we1j0)

    for l in range(N_LAYERS):
        p = layers[l]
        _, _, wd, we1e = wsplit(p)
        xpl = jnp.transpose(xp[:, :3])
        ar, br, xrel = _sc_gather(a, b, xpl, rcv2, src2)
        m0, m1, m2, m3, rc = _tc_edge(
            ar, br, xrel, pp_edge_attr,
            we1e, wd, p['be1'].reshape(1, DIM_H),
            p['We2'], p['be2'].reshape(1, DIM_H),
            p['Wx'].reshape(1, DIM_H), p['bx'].reshape(1, 1))
        if l < N_LAYERS - 1:
            g0, g1, g2, g3, gx = _sc_scatter(m0, m1, m2, m3, rc, rcv2)
        else:
            g0, g1, g2, g3 = _sc_scatter4(m0, m1, m2, m3, rcv2)
            gx = gx0
        if l == 0:
            gx0 = gx
        wh1 = p['Wh1']
        if l < N_LAYERS - 1:
            we1i_n, we1j_n, _, _ = wsplit(layers[l + 1])
            h, xp, a, b = _tc_node(
                h, xp, (g0, g1, g2, g3), gx,
                wh1[0:DIM_H], wh1[DIM_H:], p['bh1'].reshape(1, DIM_H),
                p['Wh2'], p['bh2'].reshape(1, DIM_H), we1i_n, we1j_n)
        else:
            h, gp = _tc_final(
                h, xp, (g0, g1, g2, g3), gx,
                wh1[0:DIM_H], wh1[DIM_H:], p['bh1'].reshape(1, DIM_H),
                p['Wh2'], p['bh2'].reshape(1, DIM_H))

    return (h, gp.reshape(DIM_H))


# speedup vs baseline: 1.1306x; 1.0220x over previous
import functools

import jax
import jax.numpy as jnp
from jax import lax
from jax.experimental import pallas as pl
from jax.experimental.pallas import tpu as pltpu
from jax.experimental.pallas import tpu_sc as plsc

N_NODES = 10000
N_EDGES = 160000
DIM_IN = 256
DIM_H = 512
DIM_E = 16
N_LAYERS = 4

SC_CORES = 2
SC_SUBCORES = 16
NW = SC_CORES * SC_SUBCORES
EPW = N_EDGES // NW
CHUNK = 40
NCHUNK = EPW // CHUNK
N_PAD = 10240
NPW = N_PAD // SC_SUBCORES
ZROWS = 128

XW = 128
BN = 1000
BE = 1000

@functools.cache
def _sc_mesh():
    return plsc.VectorSubcoreMesh(core_axis_name="c", subcore_axis_name="s",
                                  num_cores=SC_CORES,
                                  num_subcores=SC_SUBCORES)


def _sc_gather_body(a_hbm, b_hbm, xpl_hbm, rcv_hbm, src_hbm,
                    ar_out, br_out, xrel_out,
                    idxr_v, idxs_v, xpl_v,
                    buf_a0, buf_b0, buf_r0,
                    buf_a1, buf_b1, buf_r1,
                    in_s0, in_s1, out_s0, out_s1):
    core = lax.axis_index("c")
    sub = lax.axis_index("s")
    wid = core * SC_SUBCORES + sub
    pltpu.sync_copy(rcv_hbm.at[wid], idxr_v)
    pltpu.sync_copy(src_hbm.at[wid], idxs_v)
    pltpu.sync_copy(xpl_hbm, xpl_v)

    sets = ((buf_a0, buf_b0, buf_r0, in_s0, out_s0),
            (buf_a1, buf_b1, buf_r1, in_s1, out_s1))

    zero16 = jnp.zeros((16,), jnp.float32)

    def zrel(i, carry):
        for _, _, br_, _, _ in sets:
            br_[i, pl.ds(0, 16)] = zero16
        return carry

    lax.fori_loop(0, CHUNK, zrel, 0)

    iota = lax.broadcasted_iota(jnp.int32, (16,), 0)
    tail_mask = iota >= 8

    def issue(j, b):
        ba, bb, _, ins, _ = sets[b]
        pltpu.async_copy(a_hbm.at[idxr_v.at[j]], ba, ins)
        pltpu.async_copy(b_hbm.at[idxs_v.at[j]], bb, ins)

    def drain_in(b):
        ba, bb, _, ins, _ = sets[b]
        pltpu.make_async_copy(a_hbm.at[pl.ds(0, CHUNK)], ba, ins).wait()
        pltpu.make_async_copy(b_hbm.at[pl.ds(0, CHUNK)], bb, ins).wait()

    def rel_compute(j, b):
        _, _, br_, _, _ = sets[b]
        for off, mask in ((0, None), (16, None), (24, tail_mask)):
            ivr = idxr_v[j, pl.ds(off, 16)]
            ivs = idxs_v[j, pl.ds(off, 16)]
            rows = iota + off
            d2 = zero16
            for c in range(3):
                cc = jnp.full((16,), c, jnp.int32)
                xr = plsc.load_gather(xpl_v, [cc, ivr])
                xs = plsc.load_gather(xpl_v, [cc, ivs])
                rel = xr - xs
                d2 = d2 + rel * rel
                plsc.store_scatter(br_, [rows, cc], rel, mask=mask)
            plsc.store_scatter(br_, [rows, jnp.full((16,), 3, jnp.int32)],
                               d2, mask=mask)

    def out_copies(j, b):
        ba, bb, br_, _, outs = sets[b]
        base = wid * EPW + j * CHUNK
        pltpu.async_copy(ba, ar_out.at[pl.ds(base, CHUNK)], outs)
        pltpu.async_copy(bb, br_out.at[pl.ds(base, CHUNK)], outs)
        pltpu.async_copy(br_, xrel_out.at[pl.ds(base, CHUNK)], outs)

    def drain_out(b):
        ba, bb, br_, _, outs = sets[b]
        pltpu.make_async_copy(ba, ar_out.at[pl.ds(0, CHUNK)], outs).wait()
        pltpu.make_async_copy(bb, br_out.at[pl.ds(0, CHUNK)], outs).wait()
        pltpu.make_async_copy(br_, xrel_out.at[pl.ds(0, CHUNK)], outs).wait()

    issue(0, 0)
    issue(1, 1)

    def body(k, carry):
        for b in range(2):
            j = 2 * k + b
            rel_compute(j, b)
            drain_in(b)
            out_copies(j, b)
            drain_out(b)

            @pl.when(j + 2 < NCHUNK)
            def _():
                issue(j + 2, b)
        return carry

    lax.fori_loop(0, NCHUNK // 2, body, 0)
    for b in range(NCHUNK - 2 * (NCHUNK // 2)):
        j = 2 * (NCHUNK // 2) + b
        rel_compute(j, b)
        drain_in(b)
        out_copies(j, b)
        drain_out(b)


@functools.cache
def _sc_gather_fn():
    return pl.kernel(
        _sc_gather_body,
        out_type=[
            jax.ShapeDtypeStruct((N_EDGES, 256), jnp.uint32),
            jax.ShapeDtypeStruct((N_EDGES, 256), jnp.uint32),
            jax.ShapeDtypeStruct((N_EDGES, DIM_E), jnp.float32),
        ],
        mesh=_sc_mesh(),
        scratch_types=[
            pltpu.VMEM((NCHUNK, CHUNK), jnp.int32),
            pltpu.VMEM((NCHUNK, CHUNK), jnp.int32),
            pltpu.VMEM((3, N_NODES), jnp.float32),
            pltpu.VMEM((CHUNK, 256), jnp.uint32),
            pltpu.VMEM((CHUNK, 256), jnp.uint32),
            pltpu.VMEM((CHUNK, DIM_E), jnp.float32),
            pltpu.VMEM((CHUNK, 256), jnp.uint32),
            pltpu.VMEM((CHUNK, 256), jnp.uint32),
            pltpu.VMEM((CHUNK, DIM_E), jnp.float32),
            pltpu.SemaphoreType.DMA,
            pltpu.SemaphoreType.DMA,
            pltpu.SemaphoreType.DMA,
            pltpu.SemaphoreType.DMA,
        ],
        compiler_params=pltpu.CompilerParams(needs_layout_passes=False),
    )


def _sc_gather(*args):
    return _sc_gather_fn()(*args)


def _sc_scatter_m_body(m0, m1, m2, m3, rc, zeros_hbm, rcv_hbm,
                       g0, g1, g2, g3, gx,
                       idx_v, mbuf0, mbuf1, mbuf2, mbuf3, mbuf4, mbuf5, acc,
                       in_s0, in_s1, in_s2, in_s3, in_s4, in_s5,
                       add_s0, add_s1, add_s2, add_s3, add_s4, add_s5):
    core = lax.axis_index("c")
    sub = lax.axis_index("s")
    wid = core * SC_SUBCORES + sub
    pltpu.sync_copy(rcv_hbm.at[wid], idx_v)

    stripe0 = sub * NPW
    sets = ((mbuf0, in_s0, add_s0), (mbuf1, in_s1, add_s1),
            (mbuf2, in_s2, add_s2), (mbuf3, in_s3, add_s3),
            (mbuf4, in_s4, add_s4), (mbuf5, in_s5, add_s5))
    NS = 6

    for m_hbm, g_out in ((m0, g0), (m1, g1), (m2, g2), (m3, g3), (rc, gx)):
        for t in range(NPW // ZROWS):
            pltpu.sync_copy(zeros_hbm,
                            acc.at[pl.ds(stripe0 + t * ZROWS, ZROWS)])
        plsc.subcore_barrier()

        def issue_in(j, b, m_hbm=m_hbm):
            buf, ins, _ = sets[b]
            base = wid * EPW + j * CHUNK
            pltpu.async_copy(m_hbm.at[pl.ds(base, CHUNK)], buf, ins)

        def drain_in(b, m_hbm=m_hbm):
            buf, ins, _ = sets[b]
            pltpu.make_async_copy(m_hbm.at[pl.ds(0, CHUNK)], buf, ins).wait()

        def add_sync(j, b):
            buf, _, adds = sets[b]
            pltpu.async_copy(buf, acc.at[idx_v.at[j]], adds, add=True).wait()

        for b in range(NS):
            issue_in(b, b)

        def body(k, carry):
            for b in range(NS):
                j = NS * k + b
                drain_in(b)
                add_sync(j, b)

                @pl.when(j + NS < NCHUNK)
                def _():
                    issue_in(j + NS, b)
            return carry

        lax.fori_loop(0, NCHUNK // NS, body, 0)
        for b in range(NCHUNK - NS * (NCHUNK // NS)):
            j = NS * (NCHUNK // NS) + b
            drain_in(b)
            add_sync(j, b)
        plsc.subcore_barrier()
        pltpu.sync_copy(acc.at[pl.ds(stripe0, NPW)],
                        g_out.at[core, pl.ds(stripe0, NPW)])
        plsc.subcore_barrier()


@functools.cache
def _sc_scatter_m_fn():
    return pl.kernel(
        _sc_scatter_m_body,
        out_type=[
            jax.ShapeDtypeStruct((SC_CORES, N_PAD, 128), jnp.float32),
            jax.ShapeDtypeStruct((SC_CORES, N_PAD, 128), jnp.float32),
            jax.ShapeDtypeStruct((SC_CORES, N_PAD, 128), jnp.float32),
            jax.ShapeDtypeStruct((SC_CORES, N_PAD, 128), jnp.float32),
            jax.ShapeDtypeStruct((SC_CORES, N_PAD, 128), jnp.float32),
        ],
        mesh=_sc_mesh(),
        scratch_types=[
            pltpu.VMEM((NCHUNK, CHUNK), jnp.int32),
            pltpu.VMEM((CHUNK, 128), jnp.float32),
            pltpu.VMEM((CHUNK, 128), jnp.float32),
            pltpu.VMEM((CHUNK, 128), jnp.float32),
            pltpu.VMEM((CHUNK, 128), jnp.float32),
            pltpu.VMEM((CHUNK, 128), jnp.float32),
            pltpu.VMEM((CHUNK, 128), jnp.float32),
            pltpu.VMEM_SHARED((N_PAD, 128), jnp.float32),
        ] + [pltpu.SemaphoreType.DMA] * 12,
    )


def _sc_scatter4_body(m0, m1, m2, m3, zeros_hbm, rcv_hbm,
                      g0, g1, g2, g3,
                      idx_v, mbuf0, mbuf1, mbuf2, mbuf3, mbuf4, mbuf5, acc,
                      in_s0, in_s1, in_s2, in_s3, in_s4, in_s5,
                      add_s0, add_s1, add_s2, add_s3, add_s4, add_s5):
    core = lax.axis_index("c")
    sub = lax.axis_index("s")
    wid = core * SC_SUBCORES + sub
    pltpu.sync_copy(rcv_hbm.at[wid], idx_v)

    stripe0 = sub * NPW
    sets = ((mbuf0, in_s0, add_s0), (mbuf1, in_s1, add_s1),
            (mbuf2, in_s2, add_s2), (mbuf3, in_s3, add_s3),
            (mbuf4, in_s4, add_s4), (mbuf5, in_s5, add_s5))
    NS = 6

    for m_hbm, g_out in ((m0, g0), (m1, g1), (m2, g2), (m3, g3)):
        for t in range(NPW // ZROWS):
            pltpu.sync_copy(zeros_hbm,
                            acc.at[pl.ds(stripe0 + t * ZROWS, ZROWS)])
        plsc.subcore_barrier()

        def issue_in(j, b, m_hbm=m_hbm):
            buf, ins, _ = sets[b]
            base = wid * EPW + j * CHUNK
            pltpu.async_copy(m_hbm.at[pl.ds(base, CHUNK)], buf, ins)

        def drain_in(b, m_hbm=m_hbm):
            buf, ins, _ = sets[b]
            pltpu.make_async_copy(m_hbm.at[pl.ds(0, CHUNK)], buf, ins).wait()

        def add_sync(j, b):
            buf, _, adds = sets[b]
            pltpu.async_copy(buf, acc.at[idx_v.at[j]], adds, add=True).wait()

        for b in range(NS):
            issue_in(b, b)

        def body(k, carry):
            for b in range(NS):
                j = NS * k + b
                drain_in(b)
                add_sync(j, b)

                @pl.when(j + NS < NCHUNK)
                def _():
                    issue_in(j + NS, b)
            return carry

        lax.fori_loop(0, NCHUNK // NS, body, 0)
        for b in range(NCHUNK - NS * (NCHUNK // NS)):
            j = NS * (NCHUNK // NS) + b
            drain_in(b)
            add_sync(j, b)
        plsc.subcore_barrier()
        pltpu.sync_copy(acc.at[pl.ds(stripe0, NPW)],
                        g_out.at[core, pl.ds(stripe0, NPW)])
        plsc.subcore_barrier()


@functools.cache
def _sc_scatter4_fn():
    return pl.kernel(
        _sc_scatter4_body,
        out_type=[
            jax.ShapeDtypeStruct((SC_CORES, N_PAD, 128), jnp.float32),
            jax.ShapeDtypeStruct((SC_CORES, N_PAD, 128), jnp.float32),
            jax.ShapeDtypeStruct((SC_CORES, N_PAD, 128), jnp.float32),
            jax.ShapeDtypeStruct((SC_CORES, N_PAD, 128), jnp.float32),
        ],
        mesh=_sc_mesh(),
        scratch_types=[
            pltpu.VMEM((NCHUNK, CHUNK), jnp.int32),
            pltpu.VMEM((CHUNK, 128), jnp.float32),
            pltpu.VMEM((CHUNK, 128), jnp.float32),
            pltpu.VMEM((CHUNK, 128), jnp.float32),
            pltpu.VMEM((CHUNK, 128), jnp.float32),
            pltpu.VMEM((CHUNK, 128), jnp.float32),
            pltpu.VMEM((CHUNK, 128), jnp.float32),
            pltpu.VMEM_SHARED((N_PAD, 128), jnp.float32),
        ] + [pltpu.SemaphoreType.DMA] * 12,
    )


def _sc_scatter4(m0, m1, m2, m3, rcv2):
    zeros = jnp.zeros((ZROWS, 128), jnp.float32)
    return _sc_scatter4_fn()(m0, m1, m2, m3, zeros, rcv2)


def _sc_scatter(m0, m1, m2, m3, rc, rcv2):
    zeros = jnp.zeros((ZROWS, 128), jnp.float32)
    return _sc_scatter_m_fn()(m0, m1, m2, m3, rc, zeros, rcv2)


def _mm(a, b):
    return jnp.dot(a, b, preferred_element_type=jnp.float32)


def _pack_half(x):
    lo = lax.bitcast_convert_type(
        x[:, :DIM_H // 2].astype(jnp.bfloat16), jnp.uint16).astype(jnp.uint32)
    hi = lax.bitcast_convert_type(
        x[:, DIM_H // 2:].astype(jnp.bfloat16), jnp.uint16).astype(jnp.uint32)
    return lo | (hi << 16)


def _unpack_half(w):
    lo = lax.bitcast_convert_type(w.astype(jnp.uint16), jnp.bfloat16)
    hi = lax.bitcast_convert_type((w >> 16).astype(jnp.uint16), jnp.bfloat16)
    return lo.astype(jnp.float32), hi.astype(jnp.float32)


def _prologue_body(feat, w_in, b_in, we1i, we1j, h0_o, a_o, b_o):
    h0 = _mm(feat[...], w_in[...]) + b_in[...]
    h0_o[...] = h0
    a_o[...] = _pack_half(_mm(h0, we1i[...]))
    b_o[...] = _pack_half(_mm(h0, we1j[...]))


def _tc_prologue(feat, w_in, b_in, we1i, we1j):
    grid = (N_NODES // BN,)
    row = pl.BlockSpec((BN, DIM_IN), lambda i: (i, 0))
    out = pl.BlockSpec((BN, DIM_H), lambda i: (i, 0))
    outT = pl.BlockSpec((BN, DIM_H // 2), lambda i: (i, 0))
    full = lambda shape: pl.BlockSpec(shape, lambda i: (0,) * len(shape))
    return pl.pallas_call(
        _prologue_body,
        grid=grid,
        in_specs=[row, full((DIM_IN, DIM_H)), full((1, DIM_H)),
                  full((DIM_H, DIM_H)), full((DIM_H, DIM_H))],
        out_specs=[out, outT, outT],
        out_shape=[jax.ShapeDtypeStruct((N_NODES, DIM_H), jnp.float32),
                   jax.ShapeDtypeStruct((N_NODES, DIM_H // 2), jnp.uint32),
                   jax.ShapeDtypeStruct((N_NODES, DIM_H // 2), jnp.uint32)],
    )(feat, w_in, b_in, we1i, we1j)


def _edge_body(ar, br, xrel, ea, we1e, wd, be1, we2, be2, wx, bx,
               m0_o, m1_o, m2_o, m3_o, rc_o):
    alo, ahi = _unpack_half(ar[...])
    blo, bhi = _unpack_half(br[...])
    pre = jnp.concatenate([alo + blo, ahi + bhi], axis=1)
    xre = xrel[...]
    d2 = xre[:, 3:4]
    t = pre + d2 * wd[...] + _mm(ea[...], we1e[...]) + be1[...]
    t = jax.nn.silu(t).astype(jnp.bfloat16)
    m = jax.nn.silu(_mm(t, we2[...].astype(jnp.bfloat16)) + be2[...])
    cw = jnp.tanh(jnp.sum(m * wx[...], axis=1, keepdims=True) + bx[...])
    col = lax.broadcasted_iota(jnp.int32, (BE, DIM_E), 1)
    rel = jnp.where(col < 3, xre, 0.0)
    rc16 = rel * cw + jnp.where(col == 3, 1.0, 0.0)
    rc = jnp.concatenate([rc16, jnp.zeros((BE, XW - DIM_E), jnp.float32)],
                         axis=1)
    m0_o[...] = m[:, 0:128]
    m1_o[...] = m[:, 128:256]
    m2_o[...] = m[:, 256:384]
    m3_o[...] = m[:, 384:512]
    rc_o[...] = rc


def _tc_edge(ar, br, xrel, ea, we1e, wd, be1, we2, be2, wx, bx):
    grid = (N_EDGES // BE,)
    rowH = pl.BlockSpec((BE, DIM_H // 2), lambda i: (i, 0))
    rowE = pl.BlockSpec((BE, DIM_E), lambda i: (i, 0))
    rowX = pl.BlockSpec((BE, XW), lambda i: (i, 0))
    full = lambda shape: pl.BlockSpec(shape, lambda i: (0,) * len(shape))
    out128 = pl.BlockSpec((BE, 128), lambda i: (i, 0))
    return pl.pallas_call(
        _edge_body,
        grid=grid,
        in_specs=[rowH, rowH, rowE, rowE,
                  full((DIM_E, DIM_H)), full((1, DIM_H)), full((1, DIM_H)),
                  full((DIM_H, DIM_H)), full((1, DIM_H)), full((1, DIM_H)),
                  full((1, 1))],
        out_specs=[out128, out128, out128, out128, out128],
        out_shape=[jax.ShapeDtypeStruct((N_EDGES, 128), jnp.float32)] * 5,
    )(ar, br, xrel, ea, we1e, wd, be1, we2, be2, wx, bx)


def _edge_body_last(ar, br, xrel, ea, we1e, wd, be1, we2, be2,
                    m0_o, m1_o, m2_o, m3_o):
    alo, ahi = _unpack_half(ar[...])
    blo, bhi = _unpack_half(br[...])
    pre = jnp.concatenate([alo + blo, ahi + bhi], axis=1)
    xre = xrel[...]
    d2 = xre[:, 3:4]
    t = pre + d2 * wd[...] + _mm(ea[...], we1e[...]) + be1[...]
    t = jax.nn.silu(t).astype(jnp.bfloat16)
    m = jax.nn.silu(_mm(t, we2[...].astype(jnp.bfloat16)) + be2[...])
    m0_o[...] = m[:, 0:128]
    m1_o[...] = m[:, 128:256]
    m2_o[...] = m[:, 256:384]
    m3_o[...] = m[:, 384:512]


def _tc_edge_last(ar, br, xrel, ea, we1e, wd, be1, we2, be2):
    grid = (N_EDGES // BE,)
    rowH = pl.BlockSpec((BE, DIM_H // 2), lambda i: (i, 0))
    rowE = pl.BlockSpec((BE, DIM_E), lambda i: (i, 0))
    full = lambda shape: pl.BlockSpec(shape, lambda i: (0,) * len(shape))
    out128 = pl.BlockSpec((BE, 128), lambda i: (i, 0))
    return pl.pallas_call(
        _edge_body_last,
        grid=grid,
        in_specs=[rowH, rowH, rowE, rowE,
                  full((DIM_E, DIM_H)), full((1, DIM_H)), full((1, DIM_H)),
                  full((DIM_H, DIM_H)), full((1, DIM_H))],
        out_specs=[out128, out128, out128, out128],
        out_shape=[jax.ShapeDtypeStruct((N_EDGES, 128), jnp.float32)] * 4,
    )(ar, br, xrel, ea, we1e, wd, be1, we2, be2)


def _node_common(h_ref, g0, g1, g2, g3, gx, wh1h, wh1a, bh1, wh2, bh2):
    h = h_ref[...]
    agg = jnp.concatenate(
        [g0[0] + g0[1], g1[0] + g1[1], g2[0] + g2[1], g3[0] + g3[1]], axis=1)
    xa = gx[0] + gx[1]
    deg = xa[:, 3:4]

    invd = 1.0 / jnp.maximum(deg, 1.0)
    agg = agg * invd
    u = jax.nn.silu(_mm(h, wh1h[...]) + _mm(agg, wh1a[...]) + bh1[...])
    h_new = h + _mm(u, wh2[...]) + bh2[...]
    return h_new, xa, invd


def _node_body(h_ref, xp, g0, g1, g2, g3, gx,
               wh1h, wh1a, bh1, wh2, bh2, we1i, we1j,
               h_o, x_o, a_o, b_o):
    h_new, xa, invd = _node_common(h_ref, g0, g1, g2, g3, gx,
                                   wh1h, wh1a, bh1, wh2, bh2)
    col = lax.broadcasted_iota(jnp.int32, (BN, XW), 1)
    mask3 = jnp.where(col < 3, 1.0, 0.0)
    x_o[...] = xp[...] + xa * invd * mask3
    h_o[...] = h_new
    a_o[...] = _pack_half(_mm(h_new, we1i[...]))
    b_o[...] = _pack_half(_mm(h_new, we1j[...]))


def _final_body(h_ref, xp, g0, g1, g2, g3, gx,
                wh1h, wh1a, bh1, wh2, bh2, h_o, gp_o):
    h_new, _, _ = _node_common(h_ref, g0, g1, g2, g3, gx,
                               wh1h, wh1a, bh1, wh2, bh2)
    h_o[...] = h_new

    @pl.when(pl.program_id(0) == 0)
    def _():
        gp_o[...] = jnp.zeros_like(gp_o)

    gp_o[...] += jnp.sum(h_new, axis=0, keepdims=True) / N_NODES


def _node_specs():
    rowH = pl.BlockSpec((BN, DIM_H), lambda i: (i, 0))
    rowE = pl.BlockSpec((BN, XW), lambda i: (i, 0))
    g128 = pl.BlockSpec((SC_CORES, BN, 128), lambda i: (0, i, 0))
    gE = g128
    full = lambda shape: pl.BlockSpec(shape, lambda i: (0,) * len(shape))
    w = full((DIM_H, DIM_H))
    b = full((1, DIM_H))
    return rowH, rowE, g128, gE, w, b


def _tc_node(h, xp, gs, gx, wh1h, wh1a, bh1, wh2, bh2, we1i, we1j):
    rowH, rowE, g128, gE, w, b = _node_specs()
    outT = pl.BlockSpec((BN, DIM_H // 2), lambda i: (i, 0))
    return pl.pallas_call(
        _node_body,
        grid=(N_NODES // BN,),
        in_specs=[rowH, rowE, g128, g128, g128, g128, gE, w, w, b, w, b, w, w],
        out_specs=[rowH, rowE, outT, outT],
        out_shape=[
            jax.ShapeDtypeStruct((N_NODES, DIM_H), jnp.float32),
            jax.ShapeDtypeStruct((N_NODES, XW), jnp.float32),
            jax.ShapeDtypeStruct((N_NODES, DIM_H // 2), jnp.uint32),
            jax.ShapeDtypeStruct((N_NODES, DIM_H // 2), jnp.uint32),
        ],
    )(h, xp, *gs, gx, wh1h, wh1a, bh1, wh2, bh2, we1i, we1j)


def _tc_final(h, xp, gs, gx, wh1h, wh1a, bh1, wh2, bh2):
    rowH, rowE, g128, gE, w, b = _node_specs()
    gp = pl.BlockSpec((1, DIM_H), lambda i: (0, 0))
    return pl.pallas_call(
        _final_body,
        grid=(N_NODES // BN,),
        in_specs=[rowH, rowE, g128, g128, g128, g128, gE, w, w, b, w, b],
        out_specs=[rowH, gp],
        out_shape=[
            jax.ShapeDtypeStruct((N_NODES, DIM_H), jnp.float32),
            jax.ShapeDtypeStruct((1, DIM_H), jnp.float32),
        ],
    )(h, xp, *gs, gx, wh1h, wh1a, bh1, wh2, bh2)


def kernel(protein_pos, protein_atom_feature, pp_edge_index, pp_edge_attr,
           params):
    src = pp_edge_index[0].astype(jnp.int32)
    rcv = pp_edge_index[1].astype(jnp.int32)
    rcv2 = rcv.reshape(NW, NCHUNK, CHUNK)
    src2 = src.reshape(NW, NCHUNK, CHUNK)
    xp = jnp.pad(protein_pos.astype(jnp.float32), ((0, 0), (0, XW - 3)))

    layers = params['layers']

    def wsplit(p):
        we1 = p['We1']
        return (we1[0:DIM_H], we1[DIM_H:2 * DIM_H],
                we1[2 * DIM_H:2 * DIM_H + 1],
                we1[2 * DIM_H + 1:])

    we1i0, we1j0, _, _ = wsplit(layers[0])
    h, a, b = _tc_prologue(
        protein_atom_feature, params['W_in'], params['b_in'].reshape(1, DIM_H),
        we1i0, we1j0)

    for l in range(N_LAYERS):
        p = layers[l]
        _, _, wd, we1e = wsplit(p)
        xpl = jnp.transpose(xp[:, :3])
        ar, br, xrel = _sc_gather(a, b, xpl, rcv2, src2)
        if l < N_LAYERS - 1:
            m0, m1, m2, m3, rc = _tc_edge(
                ar, br, xrel, pp_edge_attr,
                we1e, wd, p['be1'].reshape(1, DIM_H),
                p['We2'], p['be2'].reshape(1, DIM_H),
                p['Wx'].reshape(1, DIM_H), p['bx'].reshape(1, 1))
            g0, g1, g2, g3, gx = _sc_scatter(m0, m1, m2, m3, rc, rcv2)
        else:
            m0, m1, m2, m3 = _tc_edge_last(
                ar, br, xrel, pp_edge_attr,
                we1e, wd, p['be1'].reshape(1, DIM_H),
                p['We2'], p['be2'].reshape(1, DIM_H))
            g0, g1, g2, g3 = _sc_scatter4(m0, m1, m2, m3, rcv2)
            gx = gx0
        if l == 0:
            gx0 = gx
        wh1 = p['Wh1']
        if l < N_LAYERS - 1:
            we1i_n, we1j_n, _, _ = wsplit(layers[l + 1])
            h, xp, a, b = _tc_node(
                h, xp, (g0, g1, g2, g3), gx,
                wh1[0:DIM_H], wh1[DIM_H:], p['bh1'].reshape(1, DIM_H),
                p['Wh2'], p['bh2'].reshape(1, DIM_H), we1i_n, we1j_n)
        else:
            h, gp = _tc_final(
                h, xp, (g0, g1, g2, g3), gx,
                wh1[0:DIM_H], wh1[DIM_H:], p['bh1'].reshape(1, DIM_H),
                p['Wh2'], p['bh2'].reshape(1, DIM_H))

    return (h, gp.reshape(DIM_H))
